# baseline (speedup 1.0000x reference)
import functools

import jax
import jax.numpy as jnp
import numpy as np
from jax import lax
from jax.experimental import pallas as pl
from jax.experimental.pallas import tpu as pltpu
from jax.experimental.pallas import tpu_sc as plsc

BB, LL, KK = 4, 1024, 30
NSD, NVD = 100, 16
ESD, EVD = 32, 1
NLAY = 4
NNODE = BB * LL
NEDGE = NNODE * KK
TW = 256
EW = 40
NBLK = 128
EBLK = NBLK * KK
NGRID = NNODE // NBLK

_HI = lax.Precision.HIGHEST
f32 = jnp.float32


def _knn_body(cand_ref, qs_ref, cols_ref, idx_ref, geo_ref, ohsc, idsc):
    b = pl.program_id(0)
    j = pl.program_id(1)
    cand = cand_ref[0]
    q = cols_ref[0]
    cx, cy, cz = cand[:, 0:1], cand[:, 1:2], cand[:, 2:3]
    rx, ry, rz = q[0:1, :], q[1:2, :], q[2:3, :]
    x2c = cx * cx + cy * cy + cz * cz
    x2r = rx * rx + ry * ry + rz * rz
    dot = cx * rx + cy * ry + cz * rz
    d2 = (x2r + x2c) - 2.0 * dot
    d2 = jnp.maximum(d2, 0.0)
    cand_ids = jax.lax.broadcasted_iota(jnp.int32, (LL, NBLK), 0)
    row_ids = jax.lax.broadcasted_iota(jnp.int32, (LL, NBLK), 1) + j * NBLK
    d2 = jnp.where(cand_ids == row_ids, d2 + 1e12, d2)

    candf = jax.lax.broadcasted_iota(jnp.int32, (LL, 1), 0).astype(f32)
    rowf = (jax.lax.broadcasted_iota(jnp.int32, (1, NBLK), 1) + j * NBLK).astype(f32)
    val = d2
    for k in range(KK):
        m = jnp.min(val, axis=0, keepdims=True)
        eq = val == m
        idxf = jnp.min(jnp.where(eq, candf, 3e9), axis=0, keepdims=True)
        oh = candf == idxf
        ohsc[:, NBLK * k:NBLK * (k + 1)] = oh.astype(f32)
        idsc[0:1, NBLK * k:NBLK * (k + 1)] = idxf
        idx_ref[0, k:k + 1, :] = (idxf + jnp.float32(1024.0) * b.astype(f32)
                                  ).astype(jnp.int32)
        val = jnp.where(oh, 1e30, val)

    qs = qs_ref[0]
    caj24 = jnp.dot(qs, ohsc[...])
    cajT = caj24[0:8, :] + caj24[8:16, :] + caj24[16:24, :]
    geoT = cajT - jnp.tile(q, (1, KK))
    offs = idsc[...] - jnp.tile(rowf, (1, KK))
    sub8 = jax.lax.broadcasted_iota(jnp.int32, (8, EBLK), 0)
    geo_ref[0] = jnp.where(sub8 == 3, offs, geoT)


def _knn_call(ca_rows, ca_splits, ca_cols):
    return pl.pallas_call(
        _knn_body,
        grid=(BB, LL // NBLK),
        in_specs=[
            pl.BlockSpec((1, LL, 8), lambda b, j: (b, 0, 0)),
            pl.BlockSpec((1, 24, LL), lambda b, j: (b, 0, 0)),
            pl.BlockSpec((1, 8, NBLK), lambda b, j: (b, 0, j)),
        ],
        out_specs=[
            pl.BlockSpec((1, KK, NBLK), lambda b, j: (b * 8 + j, 0, 0)),
            pl.BlockSpec((1, 8, EBLK), lambda b, j: (b * 8 + j, 0, 0)),
        ],
        out_shape=[
            jax.ShapeDtypeStruct((NGRID, KK, NBLK), jnp.int32),
            jax.ShapeDtypeStruct((NGRID, 8, EBLK), f32),
        ],
        scratch_shapes=[
            pltpu.VMEM((LL, EBLK), f32),
            pltpu.VMEM((1, EBLK), f32),
        ],
    )(ca_rows, ca_splits, ca_cols)


_SC_NW = 32
_SC_BPW = NEDGE // _SC_NW
_SC_CH = 128
_SC_NCH = _SC_BPW // _SC_CH


def _sc_gather(table, idx):
    mesh = plsc.VectorSubcoreMesh(core_axis_name="c", subcore_axis_name="s")

    @functools.partial(
        pl.kernel,
        mesh=mesh,
        out_type=jax.ShapeDtypeStruct((NEDGE, TW), f32),
        scratch_types=[
            pltpu.VMEM((_SC_CH,), jnp.int32),
            pltpu.VMEM((_SC_CH,), jnp.int32),
            pltpu.VMEM((_SC_CH, TW), f32),
            pltpu.VMEM((_SC_CH, TW), f32),
            pltpu.SemaphoreType.DMA,
            pltpu.SemaphoreType.DMA,
        ],
    )
    def k(table_hbm, idx_hbm, out_hbm, idx0, idx1, rows0, rows1, sem0, sem1):
        wid = lax.axis_index("s") * 2 + lax.axis_index("c")
        base = wid * _SC_BPW

        @pl.loop(0, _SC_NCH, step=2)
        def _(c):
            off0 = base + c * _SC_CH
            off1 = off0 + _SC_CH
            pltpu.sync_copy(idx_hbm.at[pl.ds(off0, _SC_CH)], idx0)
            cp0 = pltpu.async_copy(table_hbm.at[idx0], rows0, sem0)
            pltpu.sync_copy(idx_hbm.at[pl.ds(off1, _SC_CH)], idx1)
            cp1 = pltpu.async_copy(table_hbm.at[idx1], rows1, sem1)
            cp0.wait()
            pltpu.sync_copy(rows0, out_hbm.at[pl.ds(off0, _SC_CH)])
            cp1.wait()
            pltpu.sync_copy(rows1, out_hbm.at[pl.ds(off1, _SC_CH)])

    return k(table, idx)


def _gvp_small(s_in, vx, vy, vz, Wh, WsS, WsV, bs, Wv, Wg, bg, final):
    vhx, vhy, vhz = jnp.dot(vx, Wh), jnp.dot(vy, Wh), jnp.dot(vz, Wh)
    vn = jnp.sqrt(vhx * vhx + vhy * vhy + vhz * vhz + 1e-8)
    so = jnp.dot(s_in, WsS) + jnp.dot(vn, WsV) + bs
    gate = jax.nn.sigmoid(jnp.dot(so, Wg) + bg)
    vox = jnp.dot(vhx, Wv) * gate
    voy = jnp.dot(vhy, Wv) * gate
    voz = jnp.dot(vhz, Wv) * gate
    if not final:
        so = jnp.maximum(so, 0.0)
    return so, vox, voy, voz


def _layernorm(s, vx, vy, vz, g, b):
    mu = jnp.mean(s, axis=-1, keepdims=True)
    var = jnp.mean((s - mu) ** 2, axis=-1, keepdims=True)
    s = (s - mu) / jnp.sqrt(var + 1e-5) * g + b
    vn = jnp.sqrt(jnp.mean(vx * vx + vy * vy + vz * vz, axis=-1, keepdims=True) + 1e-8)
    return s, vx / vn, vy / vn, vz / vn


def _conv_body(*refs):
    (o_ref, g_ref, e_ref,
     WG, WE, WN, WsVn, bs0,
     Wv0, Wg0, bg0,
     Wh1, Ws1S, Ws1V, bs1, Wv1, Wg1, bg1,
     Wh2, Ws2S, Ws2V, bs2, Wv2, Wg2, bg2,
     g1, b1,
     Fh0, F0S, F0V, fb0, Fv0, Fg0, fg0,
     Fh1, F1S, F1V, fb1, Fv1, Fg1, fg1,
     g2, b2,
     out_ref) = refs

    nodes = o_ref[...]
    s = nodes[:, 0:NSD]
    nvx = nodes[:, 100:116]
    nvy = nodes[:, 116:132]
    nvz = nodes[:, 132:148]
    g = g_ref[...]
    e = e_ref[...]

    def rep(t):
        return jnp.broadcast_to(t[None], (KK,) + t.shape).reshape(EBLK, t.shape[-1])

    def kmean(x):
        return x.reshape(KK, NBLK, x.shape[-1]).sum(axis=0) / jnp.float32(KK)

    X = (jnp.dot(g, WG[...]) + jnp.dot(e, WE[...])
         + rep(jnp.dot(nodes, WN[...])))
    vhx = X[:, 100:133]
    vhy = X[:, 133:166]
    vhz = X[:, 166:199]
    vn = jnp.sqrt(vhx * vhx + vhy * vhy + vhz * vhz + 1e-8)
    so = X[:, 0:100] + jnp.dot(vn, WsVn[...]) + bs0[...]
    gate = jax.nn.sigmoid(jnp.dot(so, Wg0[...]) + bg0[...])
    mvx = jnp.dot(vhx, Wv0[...]) * gate
    mvy = jnp.dot(vhy, Wv0[...]) * gate
    mvz = jnp.dot(vhz, Wv0[...]) * gate
    ms = jnp.maximum(so, 0.0)

    ms, mvx, mvy, mvz = _gvp_small(ms, mvx, mvy, mvz, Wh1[...], Ws1S[...],
                                   Ws1V[...], bs1[...], Wv1[...], Wg1[...],
                                   bg1[...], final=False)
    ms, mvx, mvy, mvz = _gvp_small(ms, mvx, mvy, mvz, Wh2[...], Ws2S[...],
                                   Ws2V[...], bs2[...], Wv2[...], Wg2[...],
                                   bg2[...], final=True)

    ags = kmean(ms)
    agvx = kmean(mvx)
    agvy = kmean(mvy)
    agvz = kmean(mvz)

    s1, vx1, vy1, vz1 = _layernorm(s + ags, nvx + agvx, nvy + agvy, nvz + agvz,
                                   g1[...], b1[...])

    fs, fvx, fvy, fvz = _gvp_small(s1, vx1, vy1, vz1, Fh0[...], F0S[...],
                                   F0V[...], fb0[...], Fv0[...], Fg0[...],
                                   fg0[...], final=False)
    fs, fvx, fvy, fvz = _gvp_small(fs, fvx, fvy, fvz, Fh1[...], F1S[...],
                                   F1V[...], fb1[...], Fv1[...], Fg1[...],
                                   fg1[...], final=True)

    s2, vx2, vy2, vz2 = _layernorm(s1 + fs, vx1 + fvx, vy1 + fvy, vz1 + fvz,
                                   g2[...], b2[...])

    out_ref[:, 0:NSD] = s2
    out_ref[:, 100:116] = vx2
    out_ref[:, 116:132] = vy2
    out_ref[:, 132:148] = vz2
    out_ref[:, 148:TW] = jnp.zeros((NBLK, TW - 148), f32)


def _conv_call(o_prev, gat, etab, wts):
    in_specs = [
        pl.BlockSpec((NBLK, TW), lambda i: (i, 0)),
        pl.BlockSpec((EBLK, TW), lambda i: (i, 0)),
        pl.BlockSpec((EBLK, EW), lambda i: (i, 0)),
    ] + [pl.BlockSpec(w.shape, lambda i, n=len(w.shape): (0,) * n) for w in wts]
    return pl.pallas_call(
        _conv_body,
        grid=(NGRID,),
        in_specs=in_specs,
        out_specs=pl.BlockSpec((NBLK, TW), lambda i: (i, 0)),
        out_shape=jax.ShapeDtypeStruct((NNODE, TW), f32),
    )(o_prev, gat, etab, *wts)


def _edge_body(geo_ref, mu_ref, fr_ref, WsRT, WsCT, WsST, WsVrT, bsT, Whs,
               Wvs, WgT, bg, lgT, lbT, out_ref):
    g8 = geo_ref[0]
    dx, dy, dz, off = g8[0:1, :], g8[1:2, :], g8[2:3, :], g8[3:4, :]
    d2 = dx * dx + dy * dy + dz * dz
    dist = jnp.sqrt(d2 + 1e-8)
    rbfT = jnp.exp(-(((dist - mu_ref[...]) / 1.25) ** 2))
    angT = off * fr_ref[...]
    caT, saT = jnp.cos(angT), jnp.sin(angT)
    ex, ey, ez = dx / dist, dy / dist, dz / dist
    wh = Whs[...]
    vhx, vhy, vhz = ex * wh, ey * wh, ez * wh
    vn = jnp.sqrt(vhx * vhx + vhy * vhy + vhz * vhz + 1e-8)
    so = (jnp.dot(WsRT[...], rbfT) + jnp.dot(WsCT[...], caT)
          + jnp.dot(WsST[...], saT) + WsVrT[...] * vn + bsT[...])
    gate = jax.nn.sigmoid(jnp.dot(WgT[...], so) + bg[...])
    wv = Wvs[...]
    vox, voy, voz = vhx * wv * gate, vhy * wv * gate, vhz * wv * gate
    so = jnp.maximum(so, 0.0)
    mu = jnp.mean(so, axis=0, keepdims=True)
    var = jnp.mean((so - mu) ** 2, axis=0, keepdims=True)
    esT = (so - mu) / jnp.sqrt(var + 1e-5) * lgT[...] + lbT[...]
    vn2 = jnp.sqrt(vox * vox + voy * voy + voz * voz + 1e-8)
    e40 = jnp.concatenate([esT, vox / vn2, voy / vn2, voz / vn2,
                           jnp.zeros((5, EBLK), f32)], axis=0)
    out_ref[...] = e40.T


def _edge_call(geo, mu_col, fr_col, wts):
    in_specs = [
        pl.BlockSpec((1, 8, EBLK), lambda i: (i, 0, 0)),
        pl.BlockSpec((16, 1), lambda i: (0, 0)),
        pl.BlockSpec((8, 1), lambda i: (0, 0)),
    ] + [pl.BlockSpec(w.shape, lambda i, n=len(w.shape): (0,) * n) for w in wts]
    return pl.pallas_call(
        _edge_body,
        grid=(NEDGE // EBLK,),
        in_specs=in_specs,
        out_specs=pl.BlockSpec((EBLK, EW), lambda i: (i, 0)),
        out_shape=jax.ShapeDtypeStruct((NEDGE, EW), f32),
    )(geo, mu_col, fr_col, *wts)


def _node_body(sin_ref, vx_ref, vy_ref, vz_ref, Wh, WsS, WsV, bs, Wv, Wg, bg,
               lg, lb, out_ref):
    so, vox, voy, voz = _gvp_small(sin_ref[...], vx_ref[...], vy_ref[...],
                                   vz_ref[...], Wh[...], WsS[...], WsV[...],
                                   bs[...], Wv[...], Wg[...], bg[...],
                                   final=False)
    s, vx, vy, vz = _layernorm(so, vox, voy, voz, lg[...], lb[...])
    nb = s.shape[0]
    out_ref[:, 0:NSD] = s
    out_ref[:, 100:116] = vx
    out_ref[:, 116:132] = vy
    out_ref[:, 132:148] = vz
    out_ref[:, 148:TW] = jnp.zeros((nb, TW - 148), f32)


def _node_call(sin24, vfx, vfy, vfz, wts):
    blk = 512
    in_specs = [
        pl.BlockSpec((blk, 24), lambda i: (i, 0)),
        pl.BlockSpec((blk, 2), lambda i: (i, 0)),
        pl.BlockSpec((blk, 2), lambda i: (i, 0)),
        pl.BlockSpec((blk, 2), lambda i: (i, 0)),
    ] + [pl.BlockSpec(w.shape, lambda i, n=len(w.shape): (0,) * n) for w in wts]
    return pl.pallas_call(
        _node_body,
        grid=(NNODE // blk,),
        in_specs=in_specs,
        out_specs=pl.BlockSpec((blk, TW), lambda i: (i, 0)),
        out_shape=jax.ShapeDtypeStruct((NNODE, TW), f32),
    )(sin24, vfx, vfy, vfz, *wts)


def _final_body(o_ref, c_ref, s_ref, vr_ref):
    nodes = o_ref[...]
    s = nodes[:, 0:NSD]
    vx = nodes[:, 100:116]
    vy = nodes[:, 116:132]
    vz = nodes[:, 132:148]
    C = c_ref[...]
    v1x, v1y, v1z = C[:, 6:7] - C[:, 3:4], C[:, 7:8] - C[:, 4:5], C[:, 8:9] - C[:, 5:6]
    v2x, v2y, v2z = C[:, 0:1] - C[:, 3:4], C[:, 1:2] - C[:, 4:5], C[:, 2:3] - C[:, 5:6]
    n1 = jnp.sqrt(v1x * v1x + v1y * v1y + v1z * v1z + 1e-8)
    e1x, e1y, e1z = v1x / n1, v1y / n1, v1z / n1
    d12 = e1x * v2x + e1y * v2y + e1z * v2z
    u2x, u2y, u2z = v2x - e1x * d12, v2y - e1y * d12, v2z - e1z * d12
    n2 = jnp.sqrt(u2x * u2x + u2y * u2y + u2z * u2z + 1e-8)
    e2x, e2y, e2z = u2x / n2, u2y / n2, u2z / n2
    e3x = e1y * e2z - e1z * e2y
    e3y = e1z * e2x - e1x * e2z
    e3z = e1x * e2y - e1y * e2x
    s_ref[...] = s
    vr_ref[:, 0:16] = vx * e1x + vy * e1y + vz * e1z
    vr_ref[:, 16:32] = vx * e2x + vy * e2y + vz * e2z
    vr_ref[:, 32:48] = vx * e3x + vy * e3y + vz * e3z


def _final_call(o4, cflat):
    blk = 512
    return pl.pallas_call(
        _final_body,
        grid=(NNODE // blk,),
        in_specs=[
            pl.BlockSpec((blk, TW), lambda i: (i, 0)),
            pl.BlockSpec((blk, 16), lambda i: (i, 0)),
        ],
        out_specs=[
            pl.BlockSpec((blk, NSD), lambda i: (i, 0)),
            pl.BlockSpec((blk, 48), lambda i: (i, 0)),
        ],
        out_shape=[
            jax.ShapeDtypeStruct((NNODE, NSD), f32),
            jax.ShapeDtypeStruct((NNODE, 48), f32),
        ],
    )(o4, cflat)


def _norm_(x, axis=-1, keepdims=False):
    return jnp.sqrt(jnp.sum(x * x, axis=axis, keepdims=keepdims) + 1e-8)


def _normalize_(x, axis=-1):
    return x / _norm_(x, axis=axis, keepdims=True)


def _dih_features(coords):
    X = coords.reshape(coords.shape[0], -1, 3)
    dX = X[:, 1:] - X[:, :-1]
    U = _normalize_(dX)
    u2, u1, u0 = U[:, :-2], U[:, 1:-1], U[:, 2:]
    n2 = _normalize_(jnp.cross(u2, u1))
    n1 = _normalize_(jnp.cross(u1, u0))
    cosD = jnp.clip(jnp.sum(n2 * n1, axis=-1), -1 + 1e-7, 1 - 1e-7)
    D = jnp.sign(jnp.sum(u2 * n1, axis=-1)) * jnp.arccos(cosD)
    D = jnp.pad(D, ((0, 0), (1, 2)))
    D = D.reshape(D.shape[0], -1, 3)
    return jnp.concatenate([jnp.cos(D), jnp.sin(D)], axis=-1)


def _orient(ca):
    fwd = _normalize_(ca[:, 1:] - ca[:, :-1])
    bwd = _normalize_(ca[:, :-1] - ca[:, 1:])
    fwd = jnp.pad(fwd, ((0, 0), (0, 1), (0, 0)))
    bwd = jnp.pad(bwd, ((0, 0), (1, 0), (0, 0)))
    return jnp.stack([fwd, bwd], axis=-2)


def _zpad_rows(w, total, off):
    out = jnp.zeros((total, w.shape[1]), f32)
    return out.at[off:off + w.shape[0]].set(w)


def _conv_weights(p):
    m0, m1, m2, fp0, fp1 = p['m0'], p['m1'], p['m2'], p['f0'], p['f1']
    Wh0, Ws0 = m0['Wh'], m0['Ws']
    WG = jnp.zeros((TW, 199), f32)
    WG = WG.at[0:100, 0:100].set(Ws0[0:100])
    WG = WG.at[100:116, 100:133].set(Wh0[0:16])
    WG = WG.at[116:132, 133:166].set(Wh0[0:16])
    WG = WG.at[132:148, 166:199].set(Wh0[0:16])
    WE = jnp.zeros((EW, 199), f32)
    WE = WE.at[0:32, 0:100].set(Ws0[100:132])
    WE = WE.at[32:33, 100:133].set(Wh0[16:17])
    WE = WE.at[33:34, 133:166].set(Wh0[16:17])
    WE = WE.at[34:35, 166:199].set(Wh0[16:17])
    WN = jnp.zeros((TW, 199), f32)
    WN = WN.at[0:100, 0:100].set(Ws0[132:232])
    WN = WN.at[100:116, 100:133].set(Wh0[17:33])
    WN = WN.at[116:132, 133:166].set(Wh0[17:33])
    WN = WN.at[132:148, 166:199].set(Wh0[17:33])
    wts = [
        WG, WE, WN,
        Ws0[232:265],
        m0['bs'][None, :],
        m0['Wv'], m0['Wg'], m0['bg'][None, :],
    ]
    for m in (m1, m2):
        wts += [m['Wh'], m['Ws'][0:100], m['Ws'][100:116], m['bs'][None, :],
                m['Wv'], m['Wg'], m['bg'][None, :]]
    wts += [p['ln1']['g'][None, :], p['ln1']['b'][None, :]]
    wts += [fp0['Wh'], fp0['Ws'][0:100], fp0['Ws'][100:132], fp0['bs'][None, :],
            fp0['Wv'], fp0['Wg'], fp0['bg'][None, :]]
    wts += [fp1['Wh'], fp1['Ws'][0:200], fp1['Ws'][200:232], fp1['bs'][None, :],
            fp1['Wv'], fp1['Wg'], fp1['bg'][None, :]]
    wts += [p['ln2']['g'][None, :], p['ln2']['b'][None, :]]
    return wts


def kernel(struc_seqs, coords, coord_mask, padding_mask, confidence, params):
    del struc_seqs, coord_mask, padding_mask
    coords = coords.astype(f32)
    ca = coords[:, :, 1, :]

    ca_rows = jnp.pad(ca, ((0, 0), (0, 0), (0, 5)))
    ca_cols = jnp.transpose(ca_rows, (0, 2, 1))
    ca_hi = ca_cols.astype(jnp.bfloat16).astype(f32)
    ca_mid = (ca_cols - ca_hi).astype(jnp.bfloat16).astype(f32)
    ca_lo = (ca_cols - ca_hi - ca_mid).astype(jnp.bfloat16).astype(f32)
    ca_splits = jnp.concatenate([ca_hi, ca_mid, ca_lo], axis=1)
    idxg, geo = _knn_call(ca_rows, ca_splits, ca_cols)
    src_idx = idxg.reshape(NEDGE)

    ep = params['embed_edge']
    mu_col = jnp.linspace(0.0, 20.0, 16, dtype=f32)[:, None]
    fr_col = jnp.exp(jnp.arange(0, 16, 2, dtype=f32) * (-np.log(10000.0) / 16))[:, None]
    e_wts = [ep['Ws'][0:16].T, ep['Ws'][16:24].T, ep['Ws'][24:32].T,
             ep['Ws'][32:33].T, ep['bs'][:, None], ep['Wh'], ep['Wv'],
             ep['Wg'].T, ep['bg'][None, :],
             params['ln_edge']['g'][:, None], params['ln_edge']['b'][:, None]]
    etab = _edge_call(geo, mu_col, fr_col, e_wts)

    dih = _dih_features(coords)
    mu_c = jnp.linspace(0.0, 1.0, 16, dtype=f32)
    conf = jnp.exp(-(((confidence[..., None] - mu_c) * 16.0) ** 2))
    sin = jnp.concatenate([dih, conf], axis=-1).reshape(NNODE, 22)
    sin24 = jnp.pad(sin, ((0, 0), (0, 2)))
    ori = _orient(ca).reshape(NNODE, 2, 3)
    vfx, vfy, vfz = ori[:, :, 0], ori[:, :, 1], ori[:, :, 2]
    npp = params['embed_node']
    n_wts = [npp['Wh'], _zpad_rows(npp['Ws'][0:22], 24, 0), npp['Ws'][22:38],
             npp['bs'][None, :], npp['Wv'], npp['Wg'], npp['bg'][None, :],
             params['ln_node']['g'][None, :], params['ln_node']['b'][None, :]]
    otab = _node_call(sin24, vfx, vfy, vfz, n_wts)

    for lp in params['layers']:
        gat = _sc_gather(otab, src_idx)
        otab = _conv_call(otab, gat, etab, _conv_weights(lp))

    cflat = jnp.pad(coords.reshape(NNODE, 9), ((0, 0), (0, 7)))
    s_out, vr = _final_call(otab, cflat)
    vrot = vr.reshape(NNODE, 3, NVD).transpose(0, 2, 1).reshape(NNODE, NVD * 3)
    return jnp.concatenate([s_out, vrot], axis=-1).reshape(BB, LL, NSD + NVD * 3)


# speedup vs baseline: 1.0104x; 1.0104x over previous
import functools

import jax
import jax.numpy as jnp
import numpy as np
from jax import lax
from jax.experimental import pallas as pl
from jax.experimental.pallas import tpu as pltpu
from jax.experimental.pallas import tpu_sc as plsc

BB, LL, KK = 4, 1024, 30
NSD, NVD = 100, 16
ESD, EVD = 32, 1
NLAY = 4
NNODE = BB * LL
NEDGE = NNODE * KK
TW = 256
EW = 40
NBLK = 128
EBLK = NBLK * KK
NGRID = NNODE // NBLK

_HI = lax.Precision.HIGHEST
f32 = jnp.float32


def _knn_body(cand_ref, qs_ref, cols_ref, idx_ref, geo_ref, ohsc, idsc):
    b = pl.program_id(0)
    j = pl.program_id(1)
    cand = cand_ref[0]
    q = cols_ref[0]
    cx, cy, cz = cand[:, 0:1], cand[:, 1:2], cand[:, 2:3]
    rx, ry, rz = q[0:1, :], q[1:2, :], q[2:3, :]
    x2c = cx * cx + cy * cy + cz * cz
    x2r = rx * rx + ry * ry + rz * rz
    dot = cx * rx + cy * ry + cz * rz
    d2 = (x2r + x2c) - 2.0 * dot
    d2 = jnp.maximum(d2, 0.0)
    cand_ids = jax.lax.broadcasted_iota(jnp.int32, (LL, NBLK), 0)
    row_ids = jax.lax.broadcasted_iota(jnp.int32, (LL, NBLK), 1) + j * NBLK
    d2 = jnp.where(cand_ids == row_ids, d2 + 1e12, d2)

    candf = jax.lax.broadcasted_iota(jnp.int32, (LL, 1), 0).astype(f32)
    rowf = (jax.lax.broadcasted_iota(jnp.int32, (1, NBLK), 1) + j * NBLK).astype(f32)
    val = d2
    for k in range(KK):
        m = jnp.min(val, axis=0, keepdims=True)
        eq = val == m
        idxf = jnp.min(jnp.where(eq, candf, 3e9), axis=0, keepdims=True)
        oh = candf == idxf
        ohsc[:, NBLK * k:NBLK * (k + 1)] = oh.astype(f32)
        idsc[0:1, NBLK * k:NBLK * (k + 1)] = idxf
        idx_ref[0, k:k + 1, :] = (idxf + jnp.float32(1024.0) * b.astype(f32)
                                  ).astype(jnp.int32)
        val = jnp.where(oh, 1e30, val)

    qs = qs_ref[0]
    caj24 = jnp.dot(qs, ohsc[...])
    cajT = caj24[0:8, :] + caj24[8:16, :] + caj24[16:24, :]
    geoT = cajT - jnp.tile(q, (1, KK))
    offs = idsc[...] - jnp.tile(rowf, (1, KK))
    sub8 = jax.lax.broadcasted_iota(jnp.int32, (8, EBLK), 0)
    geo_ref[0] = jnp.where(sub8 == 3, offs, geoT)


def _knn_call(ca_rows, ca_splits, ca_cols):
    return pl.pallas_call(
        _knn_body,
        grid=(BB, LL // NBLK),
        in_specs=[
            pl.BlockSpec((1, LL, 8), lambda b, j: (b, 0, 0)),
            pl.BlockSpec((1, 24, LL), lambda b, j: (b, 0, 0)),
            pl.BlockSpec((1, 8, NBLK), lambda b, j: (b, 0, j)),
        ],
        out_specs=[
            pl.BlockSpec((1, KK, NBLK), lambda b, j: (b * 8 + j, 0, 0)),
            pl.BlockSpec((1, 8, EBLK), lambda b, j: (b * 8 + j, 0, 0)),
        ],
        out_shape=[
            jax.ShapeDtypeStruct((NGRID, KK, NBLK), jnp.int32),
            jax.ShapeDtypeStruct((NGRID, 8, EBLK), f32),
        ],
        scratch_shapes=[
            pltpu.VMEM((LL, EBLK), f32),
            pltpu.VMEM((1, EBLK), f32),
        ],
    )(ca_rows, ca_splits, ca_cols)


_SC_NW = 32
_SC_BPW = NEDGE // _SC_NW
_SC_CH = 128
_SC_NCH = _SC_BPW // _SC_CH


def _sc_gather(table, idx):
    mesh = plsc.VectorSubcoreMesh(core_axis_name="c", subcore_axis_name="s")

    @functools.partial(
        pl.kernel,
        mesh=mesh,
        out_type=jax.ShapeDtypeStruct((NEDGE, TW), f32),
        scratch_types=[
            pltpu.VMEM((_SC_CH,), jnp.int32),
            pltpu.VMEM((_SC_CH,), jnp.int32),
            pltpu.VMEM((_SC_CH, TW), f32),
            pltpu.VMEM((_SC_CH, TW), f32),
            pltpu.SemaphoreType.DMA,
            pltpu.SemaphoreType.DMA,
        ],
    )
    def k(table_hbm, idx_hbm, out_hbm, idx0, idx1, rows0, rows1, sem0, sem1):
        wid = lax.axis_index("s") * 2 + lax.axis_index("c")
        base = wid * _SC_BPW

        @pl.loop(0, _SC_NCH, step=2)
        def _(c):
            off0 = base + c * _SC_CH
            off1 = off0 + _SC_CH
            pltpu.sync_copy(idx_hbm.at[pl.ds(off0, _SC_CH)], idx0)
            cp0 = pltpu.async_copy(table_hbm.at[idx0], rows0, sem0)
            pltpu.sync_copy(idx_hbm.at[pl.ds(off1, _SC_CH)], idx1)
            cp1 = pltpu.async_copy(table_hbm.at[idx1], rows1, sem1)
            cp0.wait()
            pltpu.sync_copy(rows0, out_hbm.at[pl.ds(off0, _SC_CH)])
            cp1.wait()
            pltpu.sync_copy(rows1, out_hbm.at[pl.ds(off1, _SC_CH)])

    return k(table, idx)


def _gvp_small(s_in, vx, vy, vz, Wh, WsS, WsV, bs, Wv, Wg, bg, final):
    vhx, vhy, vhz = jnp.dot(vx, Wh), jnp.dot(vy, Wh), jnp.dot(vz, Wh)
    vn = jnp.sqrt(vhx * vhx + vhy * vhy + vhz * vhz + 1e-8)
    so = jnp.dot(s_in, WsS) + jnp.dot(vn, WsV) + bs
    gate = jax.nn.sigmoid(jnp.dot(so, Wg) + bg)
    vox = jnp.dot(vhx, Wv) * gate
    voy = jnp.dot(vhy, Wv) * gate
    voz = jnp.dot(vhz, Wv) * gate
    if not final:
        so = jnp.maximum(so, 0.0)
    return so, vox, voy, voz


def _layernorm(s, vx, vy, vz, g, b):
    mu = jnp.mean(s, axis=-1, keepdims=True)
    var = jnp.mean((s - mu) ** 2, axis=-1, keepdims=True)
    s = (s - mu) / jnp.sqrt(var + 1e-5) * g + b
    vn = jnp.sqrt(jnp.mean(vx * vx + vy * vy + vz * vz, axis=-1, keepdims=True) + 1e-8)
    return s, vx / vn, vy / vn, vz / vn


def _conv_body(*refs):
    (o_ref, g_ref, e_ref,
     WsG, WsE, WsD, WsVn, bs0, WhGx, WhGy, WhGz, WhEx, WhEy, WhEz, WhD,
     Wv0, Wg0, bg0,
     Wh1, Ws1S, Ws1V, bs1, Wv1, Wg1, bg1,
     Wh2, Ws2S, Ws2V, bs2, Wv2, Wg2, bg2,
     g1, b1,
     Fh0, F0S, F0V, fb0, Fv0, Fg0, fg0,
     Fh1, F1S, F1V, fb1, Fv1, Fg1, fg1,
     g2, b2,
     out_ref) = refs

    nodes = o_ref[...]
    s = nodes[:, 0:NSD]
    nvx = nodes[:, 100:116]
    nvy = nodes[:, 116:132]
    nvz = nodes[:, 132:148]
    g = g_ref[...]
    e = e_ref[...]

    def rep(t):
        return jnp.broadcast_to(t[None], (KK,) + t.shape).reshape(EBLK, t.shape[-1])

    def kmean(x):
        return x.reshape(KK, NBLK, x.shape[-1]).sum(axis=0) / jnp.float32(KK)

    vhx = jnp.dot(g, WhGx[...]) + jnp.dot(e, WhEx[...]) + rep(jnp.dot(nvx, WhD[...]))
    vhy = jnp.dot(g, WhGy[...]) + jnp.dot(e, WhEy[...]) + rep(jnp.dot(nvy, WhD[...]))
    vhz = jnp.dot(g, WhGz[...]) + jnp.dot(e, WhEz[...]) + rep(jnp.dot(nvz, WhD[...]))
    vn = jnp.sqrt(vhx * vhx + vhy * vhy + vhz * vhz + 1e-8)
    so = (jnp.dot(g, WsG[...]) + jnp.dot(e, WsE[...])
          + rep(jnp.dot(s, WsD[...])) + jnp.dot(vn, WsVn[...]) + bs0[...])
    gate = jax.nn.sigmoid(jnp.dot(so, Wg0[...]) + bg0[...])
    mvx = jnp.dot(vhx, Wv0[...]) * gate
    mvy = jnp.dot(vhy, Wv0[...]) * gate
    mvz = jnp.dot(vhz, Wv0[...]) * gate
    ms = jnp.maximum(so, 0.0)

    ms, mvx, mvy, mvz = _gvp_small(ms, mvx, mvy, mvz, Wh1[...], Ws1S[...],
                                   Ws1V[...], bs1[...], Wv1[...], Wg1[...],
                                   bg1[...], final=False)
    ms, mvx, mvy, mvz = _gvp_small(ms, mvx, mvy, mvz, Wh2[...], Ws2S[...],
                                   Ws2V[...], bs2[...], Wv2[...], Wg2[...],
                                   bg2[...], final=True)

    ags = kmean(ms)
    agvx = kmean(mvx)
    agvy = kmean(mvy)
    agvz = kmean(mvz)

    s1, vx1, vy1, vz1 = _layernorm(s + ags, nvx + agvx, nvy + agvy, nvz + agvz,
                                   g1[...], b1[...])

    fs, fvx, fvy, fvz = _gvp_small(s1, vx1, vy1, vz1, Fh0[...], F0S[...],
                                   F0V[...], fb0[...], Fv0[...], Fg0[...],
                                   fg0[...], final=False)
    fs, fvx, fvy, fvz = _gvp_small(fs, fvx, fvy, fvz, Fh1[...], F1S[...],
                                   F1V[...], fb1[...], Fv1[...], Fg1[...],
                                   fg1[...], final=True)

    s2, vx2, vy2, vz2 = _layernorm(s1 + fs, vx1 + fvx, vy1 + fvy, vz1 + fvz,
                                   g2[...], b2[...])

    out_ref[:, 0:NSD] = s2
    out_ref[:, 100:116] = vx2
    out_ref[:, 116:132] = vy2
    out_ref[:, 132:148] = vz2
    out_ref[:, 148:TW] = jnp.zeros((NBLK, TW - 148), f32)


def _conv_call(o_prev, gat, etab, wts):
    in_specs = [
        pl.BlockSpec((NBLK, TW), lambda i: (i, 0)),
        pl.BlockSpec((EBLK, TW), lambda i: (i, 0)),
        pl.BlockSpec((EBLK, EW), lambda i: (i, 0)),
    ] + [pl.BlockSpec(w.shape, lambda i, n=len(w.shape): (0,) * n) for w in wts]
    return pl.pallas_call(
        _conv_body,
        grid=(NGRID,),
        in_specs=in_specs,
        out_specs=pl.BlockSpec((NBLK, TW), lambda i: (i, 0)),
        out_shape=jax.ShapeDtypeStruct((NNODE, TW), f32),
    )(o_prev, gat, etab, *wts)


def _edge_body(geo_ref, mu_ref, fr_ref, WsRT, WsCT, WsST, WsVrT, bsT, Whs,
               Wvs, WgT, bg, lgT, lbT, out_ref):
    g8 = geo_ref[0]
    dx, dy, dz, off = g8[0:1, :], g8[1:2, :], g8[2:3, :], g8[3:4, :]
    d2 = dx * dx + dy * dy + dz * dz
    dist = jnp.sqrt(d2 + 1e-8)
    rbfT = jnp.exp(-(((dist - mu_ref[...]) / 1.25) ** 2))
    angT = off * fr_ref[...]
    caT, saT = jnp.cos(angT), jnp.sin(angT)
    ex, ey, ez = dx / dist, dy / dist, dz / dist
    wh = Whs[...]
    vhx, vhy, vhz = ex * wh, ey * wh, ez * wh
    vn = jnp.sqrt(vhx * vhx + vhy * vhy + vhz * vhz + 1e-8)
    so = (jnp.dot(WsRT[...], rbfT) + jnp.dot(WsCT[...], caT)
          + jnp.dot(WsST[...], saT) + WsVrT[...] * vn + bsT[...])
    gate = jax.nn.sigmoid(jnp.dot(WgT[...], so) + bg[...])
    wv = Wvs[...]
    vox, voy, voz = vhx * wv * gate, vhy * wv * gate, vhz * wv * gate
    so = jnp.maximum(so, 0.0)
    mu = jnp.mean(so, axis=0, keepdims=True)
    var = jnp.mean((so - mu) ** 2, axis=0, keepdims=True)
    esT = (so - mu) / jnp.sqrt(var + 1e-5) * lgT[...] + lbT[...]
    vn2 = jnp.sqrt(vox * vox + voy * voy + voz * voz + 1e-8)
    e40 = jnp.concatenate([esT, vox / vn2, voy / vn2, voz / vn2,
                           jnp.zeros((5, EBLK), f32)], axis=0)
    out_ref[...] = e40.T


def _edge_call(geo, mu_col, fr_col, wts):
    in_specs = [
        pl.BlockSpec((1, 8, EBLK), lambda i: (i, 0, 0)),
        pl.BlockSpec((16, 1), lambda i: (0, 0)),
        pl.BlockSpec((8, 1), lambda i: (0, 0)),
    ] + [pl.BlockSpec(w.shape, lambda i, n=len(w.shape): (0,) * n) for w in wts]
    return pl.pallas_call(
        _edge_body,
        grid=(NEDGE // EBLK,),
        in_specs=in_specs,
        out_specs=pl.BlockSpec((EBLK, EW), lambda i: (i, 0)),
        out_shape=jax.ShapeDtypeStruct((NEDGE, EW), f32),
    )(geo, mu_col, fr_col, *wts)


def _node_body(sin_ref, vx_ref, vy_ref, vz_ref, Wh, WsS, WsV, bs, Wv, Wg, bg,
               lg, lb, out_ref):
    so, vox, voy, voz = _gvp_small(sin_ref[...], vx_ref[...], vy_ref[...],
                                   vz_ref[...], Wh[...], WsS[...], WsV[...],
                                   bs[...], Wv[...], Wg[...], bg[...],
                                   final=False)
    s, vx, vy, vz = _layernorm(so, vox, voy, voz, lg[...], lb[...])
    nb = s.shape[0]
    out_ref[:, 0:NSD] = s
    out_ref[:, 100:116] = vx
    out_ref[:, 116:132] = vy
    out_ref[:, 132:148] = vz
    out_ref[:, 148:TW] = jnp.zeros((nb, TW - 148), f32)


def _node_call(sin24, vfx, vfy, vfz, wts):
    blk = 512
    in_specs = [
        pl.BlockSpec((blk, 24), lambda i: (i, 0)),
        pl.BlockSpec((blk, 2), lambda i: (i, 0)),
        pl.BlockSpec((blk, 2), lambda i: (i, 0)),
        pl.BlockSpec((blk, 2), lambda i: (i, 0)),
    ] + [pl.BlockSpec(w.shape, lambda i, n=len(w.shape): (0,) * n) for w in wts]
    return pl.pallas_call(
        _node_body,
        grid=(NNODE // blk,),
        in_specs=in_specs,
        out_specs=pl.BlockSpec((blk, TW), lambda i: (i, 0)),
        out_shape=jax.ShapeDtypeStruct((NNODE, TW), f32),
    )(sin24, vfx, vfy, vfz, *wts)


def _final_body(o_ref, c_ref, s_ref, vr_ref):
    nodes = o_ref[...]
    s = nodes[:, 0:NSD]
    vx = nodes[:, 100:116]
    vy = nodes[:, 116:132]
    vz = nodes[:, 132:148]
    C = c_ref[...]
    v1x, v1y, v1z = C[:, 6:7] - C[:, 3:4], C[:, 7:8] - C[:, 4:5], C[:, 8:9] - C[:, 5:6]
    v2x, v2y, v2z = C[:, 0:1] - C[:, 3:4], C[:, 1:2] - C[:, 4:5], C[:, 2:3] - C[:, 5:6]
    n1 = jnp.sqrt(v1x * v1x + v1y * v1y + v1z * v1z + 1e-8)
    e1x, e1y, e1z = v1x / n1, v1y / n1, v1z / n1
    d12 = e1x * v2x + e1y * v2y + e1z * v2z
    u2x, u2y, u2z = v2x - e1x * d12, v2y - e1y * d12, v2z - e1z * d12
    n2 = jnp.sqrt(u2x * u2x + u2y * u2y + u2z * u2z + 1e-8)
    e2x, e2y, e2z = u2x / n2, u2y / n2, u2z / n2
    e3x = e1y * e2z - e1z * e2y
    e3y = e1z * e2x - e1x * e2z
    e3z = e1x * e2y - e1y * e2x
    s_ref[...] = s
    vr_ref[:, 0:16] = vx * e1x + vy * e1y + vz * e1z
    vr_ref[:, 16:32] = vx * e2x + vy * e2y + vz * e2z
    vr_ref[:, 32:48] = vx * e3x + vy * e3y + vz * e3z


def _final_call(o4, cflat):
    blk = 512
    return pl.pallas_call(
        _final_body,
        grid=(NNODE // blk,),
        in_specs=[
            pl.BlockSpec((blk, TW), lambda i: (i, 0)),
            pl.BlockSpec((blk, 16), lambda i: (i, 0)),
        ],
        out_specs=[
            pl.BlockSpec((blk, NSD), lambda i: (i, 0)),
            pl.BlockSpec((blk, 48), lambda i: (i, 0)),
        ],
        out_shape=[
            jax.ShapeDtypeStruct((NNODE, NSD), f32),
            jax.ShapeDtypeStruct((NNODE, 48), f32),
        ],
    )(o4, cflat)


def _norm_(x, axis=-1, keepdims=False):
    return jnp.sqrt(jnp.sum(x * x, axis=axis, keepdims=keepdims) + 1e-8)


def _normalize_(x, axis=-1):
    return x / _norm_(x, axis=axis, keepdims=True)


def _dih_features(coords):
    X = coords.reshape(coords.shape[0], -1, 3)
    dX = X[:, 1:] - X[:, :-1]
    U = _normalize_(dX)
    u2, u1, u0 = U[:, :-2], U[:, 1:-1], U[:, 2:]
    n2 = _normalize_(jnp.cross(u2, u1))
    n1 = _normalize_(jnp.cross(u1, u0))
    cosD = jnp.clip(jnp.sum(n2 * n1, axis=-1), -1 + 1e-7, 1 - 1e-7)
    D = jnp.sign(jnp.sum(u2 * n1, axis=-1)) * jnp.arccos(cosD)
    D = jnp.pad(D, ((0, 0), (1, 2)))
    D = D.reshape(D.shape[0], -1, 3)
    return jnp.concatenate([jnp.cos(D), jnp.sin(D)], axis=-1)


def _orient(ca):
    fwd = _normalize_(ca[:, 1:] - ca[:, :-1])
    bwd = _normalize_(ca[:, :-1] - ca[:, 1:])
    fwd = jnp.pad(fwd, ((0, 0), (0, 1), (0, 0)))
    bwd = jnp.pad(bwd, ((0, 0), (1, 0), (0, 0)))
    return jnp.stack([fwd, bwd], axis=-2)


def _zpad_rows(w, total, off):
    out = jnp.zeros((total, w.shape[1]), f32)
    return out.at[off:off + w.shape[0]].set(w)


def _conv_weights(p):
    m0, m1, m2, fp0, fp1 = p['m0'], p['m1'], p['m2'], p['f0'], p['f1']
    Wh0, Ws0 = m0['Wh'], m0['Ws']
    wts = [
        _zpad_rows(Ws0[0:100], TW, 0),
        _zpad_rows(Ws0[100:132], EW, 0),
        Ws0[132:232],
        Ws0[232:265],
        m0['bs'][None, :],
        _zpad_rows(Wh0[0:16], TW, 100),
        _zpad_rows(Wh0[0:16], TW, 116),
        _zpad_rows(Wh0[0:16], TW, 132),
        _zpad_rows(Wh0[16:17], EW, 32),
        _zpad_rows(Wh0[16:17], EW, 33),
        _zpad_rows(Wh0[16:17], EW, 34),
        Wh0[17:33],
        m0['Wv'], m0['Wg'], m0['bg'][None, :],
    ]
    for m in (m1, m2):
        wts += [m['Wh'], m['Ws'][0:100], m['Ws'][100:116], m['bs'][None, :],
                m['Wv'], m['Wg'], m['bg'][None, :]]
    wts += [p['ln1']['g'][None, :], p['ln1']['b'][None, :]]
    wts += [fp0['Wh'], fp0['Ws'][0:100], fp0['Ws'][100:132], fp0['bs'][None, :],
            fp0['Wv'], fp0['Wg'], fp0['bg'][None, :]]
    wts += [fp1['Wh'], fp1['Ws'][0:200], fp1['Ws'][200:232], fp1['bs'][None, :],
            fp1['Wv'], fp1['Wg'], fp1['bg'][None, :]]
    wts += [p['ln2']['g'][None, :], p['ln2']['b'][None, :]]
    return wts


def kernel(struc_seqs, coords, coord_mask, padding_mask, confidence, params):
    del struc_seqs, coord_mask, padding_mask
    coords = coords.astype(f32)
    ca = coords[:, :, 1, :]

    ca_rows = jnp.pad(ca, ((0, 0), (0, 0), (0, 5)))
    ca_cols = jnp.transpose(ca_rows, (0, 2, 1))
    ca_hi = ca_cols.astype(jnp.bfloat16).astype(f32)
    ca_mid = (ca_cols - ca_hi).astype(jnp.bfloat16).astype(f32)
    ca_lo = (ca_cols - ca_hi - ca_mid).astype(jnp.bfloat16).astype(f32)
    ca_splits = jnp.concatenate([ca_hi, ca_mid, ca_lo], axis=1)
    idxg, geo = _knn_call(ca_rows, ca_splits, ca_cols)
    src_idx = idxg.reshape(NEDGE)

    ep = params['embed_edge']
    mu_col = jnp.linspace(0.0, 20.0, 16, dtype=f32)[:, None]
    fr_col = jnp.exp(jnp.arange(0, 16, 2, dtype=f32) * (-np.log(10000.0) / 16))[:, None]
    e_wts = [ep['Ws'][0:16].T, ep['Ws'][16:24].T, ep['Ws'][24:32].T,
             ep['Ws'][32:33].T, ep['bs'][:, None], ep['Wh'], ep['Wv'],
             ep['Wg'].T, ep['bg'][None, :],
             params['ln_edge']['g'][:, None], params['ln_edge']['b'][:, None]]
    etab = _edge_call(geo, mu_col, fr_col, e_wts)

    dih = _dih_features(coords)
    mu_c = jnp.linspace(0.0, 1.0, 16, dtype=f32)
    conf = jnp.exp(-(((confidence[..., None] - mu_c) * 16.0) ** 2))
    sin = jnp.concatenate([dih, conf], axis=-1).reshape(NNODE, 22)
    sin24 = jnp.pad(sin, ((0, 0), (0, 2)))
    ori = _orient(ca).reshape(NNODE, 2, 3)
    vfx, vfy, vfz = ori[:, :, 0], ori[:, :, 1], ori[:, :, 2]
    npp = params['embed_node']
    n_wts = [npp['Wh'], _zpad_rows(npp['Ws'][0:22], 24, 0), npp['Ws'][22:38],
             npp['bs'][None, :], npp['Wv'], npp['Wg'], npp['bg'][None, :],
             params['ln_node']['g'][None, :], params['ln_node']['b'][None, :]]
    otab = _node_call(sin24, vfx, vfy, vfz, n_wts)

    for lp in params['layers']:
        gat = _sc_gather(otab, src_idx)
        otab = _conv_call(otab, gat, etab, _conv_weights(lp))

    cflat = jnp.pad(coords.reshape(NNODE, 9), ((0, 0), (0, 7)))
    s_out, vr = _final_call(otab, cflat)
    vrot = vr.reshape(NNODE, 3, NVD).transpose(0, 2, 1).reshape(NNODE, NVD * 3)
    return jnp.concatenate([s_out, vrot], axis=-1).reshape(BB, LL, NSD + NVD * 3)


# speedup vs baseline: 1.0106x; 1.0002x over previous
import functools

import jax
import jax.numpy as jnp
import numpy as np
from jax import lax
from jax.experimental import pallas as pl
from jax.experimental.pallas import tpu as pltpu
from jax.experimental.pallas import tpu_sc as plsc

BB, LL, KK = 4, 1024, 30
NSD, NVD = 100, 16
ESD, EVD = 32, 1
NLAY = 4
NNODE = BB * LL
NEDGE = NNODE * KK
TW = 256
EW = 40
NBLK = 128
EBLK = NBLK * KK
NGRID = NNODE // NBLK

f32 = jnp.float32


def _knn_body(cand_ref, qs_ref, cols_ref, idx_ref, geo_ref, ohsc, idsc):
    b = pl.program_id(0)
    j = pl.program_id(1)
    cand = cand_ref[0]
    q = cols_ref[0]
    cx, cy, cz = cand[:, 0:1], cand[:, 1:2], cand[:, 2:3]
    rx, ry, rz = q[0:1, :], q[1:2, :], q[2:3, :]
    x2c = cx * cx + cy * cy + cz * cz
    x2r = rx * rx + ry * ry + rz * rz
    dot = cx * rx + cy * ry + cz * rz
    d2 = (x2r + x2c) - 2.0 * dot
    d2 = jnp.maximum(d2, 0.0)
    cand_ids = jax.lax.broadcasted_iota(jnp.int32, (LL, NBLK), 0)
    row_ids = jax.lax.broadcasted_iota(jnp.int32, (LL, NBLK), 1) + j * NBLK
    d2 = jnp.where(cand_ids == row_ids, d2 + 1e12, d2)

    candf = jax.lax.broadcasted_iota(jnp.int32, (LL, 1), 0).astype(f32)
    rowf = (jax.lax.broadcasted_iota(jnp.int32, (1, NBLK), 1) + j * NBLK).astype(f32)
    val = d2
    for k in range(KK):
        m = jnp.min(val, axis=0, keepdims=True)
        eq = val == m
        idxf = jnp.min(jnp.where(eq, candf, 3e9), axis=0, keepdims=True)
        oh = candf == idxf
        ohsc[:, NBLK * k:NBLK * (k + 1)] = oh.astype(f32)
        idsc[0:1, NBLK * k:NBLK * (k + 1)] = idxf
        idx_ref[0, k:k + 1, :] = (idxf + jnp.float32(1024.0) * b.astype(f32)
                                  ).astype(jnp.int32)
        val = jnp.where(oh, 1e30, val)

    qs = qs_ref[0]
    caj24 = jnp.dot(qs, ohsc[...])
    cajT = caj24[0:8, :] + caj24[8:16, :] + caj24[16:24, :]
    geoT = cajT - jnp.tile(q, (1, KK))
    offs = idsc[...] - jnp.tile(rowf, (1, KK))
    sub8 = jax.lax.broadcasted_iota(jnp.int32, (8, EBLK), 0)
    geo_ref[0] = jnp.where(sub8 == 3, offs, geoT)


def _knn_call(ca_rows, ca_splits, ca_cols):
    return pl.pallas_call(
        _knn_body,
        grid=(BB, LL // NBLK),
        in_specs=[
            pl.BlockSpec((1, LL, 8), lambda b, j: (b, 0, 0)),
            pl.BlockSpec((1, 24, LL), lambda b, j: (b, 0, 0)),
            pl.BlockSpec((1, 8, NBLK), lambda b, j: (b, 0, j)),
        ],
        out_specs=[
            pl.BlockSpec((1, KK, NBLK), lambda b, j: (b * 8 + j, 0, 0)),
            pl.BlockSpec((1, 8, EBLK), lambda b, j: (b * 8 + j, 0, 0)),
        ],
        out_shape=[
            jax.ShapeDtypeStruct((NGRID, KK, NBLK), jnp.int32),
            jax.ShapeDtypeStruct((NGRID, 8, EBLK), f32),
        ],
        scratch_shapes=[
            pltpu.VMEM((LL, EBLK), f32),
            pltpu.VMEM((1, EBLK), f32),
        ],
    )(ca_rows, ca_splits, ca_cols)


_SC_NW = 32
_SC_BPW = NEDGE // _SC_NW
_SC_CH = 128
_SC_NCH = _SC_BPW // _SC_CH


def _sc_gather(table, idx):
    mesh = plsc.VectorSubcoreMesh(core_axis_name="c", subcore_axis_name="s")

    @functools.partial(
        pl.kernel,
        mesh=mesh,
        out_type=jax.ShapeDtypeStruct((NEDGE, TW), f32),
        scratch_types=[
            pltpu.VMEM((_SC_CH,), jnp.int32),
            pltpu.VMEM((_SC_CH,), jnp.int32),
            pltpu.VMEM((_SC_CH, TW), f32),
            pltpu.VMEM((_SC_CH, TW), f32),
            pltpu.SemaphoreType.DMA,
            pltpu.SemaphoreType.DMA,
        ],
    )
    def k(table_hbm, idx_hbm, out_hbm, idx0, idx1, rows0, rows1, sem0, sem1):
        wid = lax.axis_index("s") * 2 + lax.axis_index("c")
        base = wid * _SC_BPW

        @pl.loop(0, _SC_NCH, step=2)
        def _(c):
            off0 = base + c * _SC_CH
            off1 = off0 + _SC_CH
            pltpu.sync_copy(idx_hbm.at[pl.ds(off0, _SC_CH)], idx0)
            cp0 = pltpu.async_copy(table_hbm.at[idx0], rows0, sem0)
            pltpu.sync_copy(idx_hbm.at[pl.ds(off1, _SC_CH)], idx1)
            cp1 = pltpu.async_copy(table_hbm.at[idx1], rows1, sem1)
            cp0.wait()
            pltpu.sync_copy(rows0, out_hbm.at[pl.ds(off0, _SC_CH)])
            cp1.wait()
            pltpu.sync_copy(rows1, out_hbm.at[pl.ds(off1, _SC_CH)])

    return k(table, idx)


def _gvp_small(s_in, vx, vy, vz, Wh, WsS, WsV, bs, Wv, Wg, bg, final):
    vhx, vhy, vhz = jnp.dot(vx, Wh), jnp.dot(vy, Wh), jnp.dot(vz, Wh)
    vn = jnp.sqrt(vhx * vhx + vhy * vhy + vhz * vhz + 1e-8)
    so = jnp.dot(s_in, WsS) + jnp.dot(vn, WsV) + bs
    gate = jax.nn.sigmoid(jnp.dot(so, Wg) + bg)
    vox = jnp.dot(vhx, Wv) * gate
    voy = jnp.dot(vhy, Wv) * gate
    voz = jnp.dot(vhz, Wv) * gate
    if not final:
        so = jnp.maximum(so, 0.0)
    return so, vox, voy, voz


def _layernorm(s, vx, vy, vz, g, b):
    mu = jnp.mean(s, axis=-1, keepdims=True)
    var = jnp.mean((s - mu) ** 2, axis=-1, keepdims=True)
    s = (s - mu) / jnp.sqrt(var + 1e-5) * g + b
    vn = jnp.sqrt(jnp.mean(vx * vx + vy * vy + vz * vz, axis=-1, keepdims=True) + 1e-8)
    return s, vx / vn, vy / vn, vz / vn


def _conv_body(*refs):
    (o_ref, g_ref, e_ref,
     WsG, WsE, WsD, WsVn, bs0, WhGx, WhGy, WhGz, WhEx, WhEy, WhEz, WhD,
     Wv0, Wg0, bg0,
     Wh1, Ws1S, Ws1V, bs1, Wv1, Wg1, bg1,
     Wh2, Ws2S, Ws2V, bs2, Wv2, Wg2, bg2,
     g1, b1,
     Fh0, F0S, F0V, fb0, Fv0, Fg0, fg0,
     Fh1, F1S, F1V, fb1, Fv1, Fg1, fg1,
     g2, b2,
     out_ref) = refs

    nodes = o_ref[...]
    s = nodes[:, 0:NSD]
    nvx = nodes[:, 100:116]
    nvy = nodes[:, 116:132]
    nvz = nodes[:, 132:148]
    g = g_ref[...]
    e = e_ref[...]

    def rep(t):
        return jnp.broadcast_to(t[None], (KK,) + t.shape).reshape(EBLK, t.shape[-1])

    def kmean(x):
        return x.reshape(KK, NBLK, x.shape[-1]).sum(axis=0) / jnp.float32(KK)

    vhx = jnp.dot(g, WhGx[...]) + jnp.dot(e, WhEx[...]) + rep(jnp.dot(nvx, WhD[...]))
    vhy = jnp.dot(g, WhGy[...]) + jnp.dot(e, WhEy[...]) + rep(jnp.dot(nvy, WhD[...]))
    vhz = jnp.dot(g, WhGz[...]) + jnp.dot(e, WhEz[...]) + rep(jnp.dot(nvz, WhD[...]))
    vn = jnp.sqrt(vhx * vhx + vhy * vhy + vhz * vhz + 1e-8)
    so = (jnp.dot(g, WsG[...]) + jnp.dot(e, WsE[...])
          + rep(jnp.dot(s, WsD[...])) + jnp.dot(vn, WsVn[...]) + bs0[...])
    gate = jax.nn.sigmoid(jnp.dot(so, Wg0[...]) + bg0[...])
    mvx = jnp.dot(vhx, Wv0[...]) * gate
    mvy = jnp.dot(vhy, Wv0[...]) * gate
    mvz = jnp.dot(vhz, Wv0[...]) * gate
    ms = jnp.maximum(so, 0.0)

    ms, mvx, mvy, mvz = _gvp_small(ms, mvx, mvy, mvz, Wh1[...], Ws1S[...],
                                   Ws1V[...], bs1[...], Wv1[...], Wg1[...],
                                   bg1[...], final=False)
    ms, mvx, mvy, mvz = _gvp_small(ms, mvx, mvy, mvz, Wh2[...], Ws2S[...],
                                   Ws2V[...], bs2[...], Wv2[...], Wg2[...],
                                   bg2[...], final=True)

    ags = kmean(ms)
    agvx = kmean(mvx)
    agvy = kmean(mvy)
    agvz = kmean(mvz)

    s1, vx1, vy1, vz1 = _layernorm(s + ags, nvx + agvx, nvy + agvy, nvz + agvz,
                                   g1[...], b1[...])

    fs, fvx, fvy, fvz = _gvp_small(s1, vx1, vy1, vz1, Fh0[...], F0S[...],
                                   F0V[...], fb0[...], Fv0[...], Fg0[...],
                                   fg0[...], final=False)
    fs, fvx, fvy, fvz = _gvp_small(fs, fvx, fvy, fvz, Fh1[...], F1S[...],
                                   F1V[...], fb1[...], Fv1[...], Fg1[...],
                                   fg1[...], final=True)

    s2, vx2, vy2, vz2 = _layernorm(s1 + fs, vx1 + fvx, vy1 + fvy, vz1 + fvz,
                                   g2[...], b2[...])

    out_ref[:, 0:NSD] = s2
    out_ref[:, 100:116] = vx2
    out_ref[:, 116:132] = vy2
    out_ref[:, 132:148] = vz2
    out_ref[:, 148:TW] = jnp.zeros((NBLK, TW - 148), f32)


def _conv_call(o_prev, gat, etab, wts):
    in_specs = [
        pl.BlockSpec((NBLK, TW), lambda i: (i, 0)),
        pl.BlockSpec((EBLK, TW), lambda i: (i, 0)),
        pl.BlockSpec((EBLK, EW), lambda i: (i, 0)),
    ] + [pl.BlockSpec(w.shape, lambda i, n=len(w.shape): (0,) * n) for w in wts]
    return pl.pallas_call(
        _conv_body,
        grid=(NGRID,),
        in_specs=in_specs,
        out_specs=pl.BlockSpec((NBLK, TW), lambda i: (i, 0)),
        out_shape=jax.ShapeDtypeStruct((NNODE, TW), f32),
    )(o_prev, gat, etab, *wts)


def _edge_body(geo_ref, mu_ref, fr_ref, WsRT, WsCT, WsST, WsVrT, bsT, Whs,
               Wvs, WgT, bg, lgT, lbT, out_ref):
    g8 = geo_ref[0]
    dx, dy, dz, off = g8[0:1, :], g8[1:2, :], g8[2:3, :], g8[3:4, :]
    d2 = dx * dx + dy * dy + dz * dz
    dist = jnp.sqrt(d2 + 1e-8)
    rbfT = jnp.exp(-(((dist - mu_ref[...]) / 1.25) ** 2))
    angT = off * fr_ref[...]
    caT, saT = jnp.cos(angT), jnp.sin(angT)
    ex, ey, ez = dx / dist, dy / dist, dz / dist
    wh = Whs[...]
    vhx, vhy, vhz = ex * wh, ey * wh, ez * wh
    vn = jnp.sqrt(vhx * vhx + vhy * vhy + vhz * vhz + 1e-8)
    so = (jnp.dot(WsRT[...], rbfT) + jnp.dot(WsCT[...], caT)
          + jnp.dot(WsST[...], saT) + WsVrT[...] * vn + bsT[...])
    gate = jax.nn.sigmoid(jnp.dot(WgT[...], so) + bg[...])
    wv = Wvs[...]
    vox, voy, voz = vhx * wv * gate, vhy * wv * gate, vhz * wv * gate
    so = jnp.maximum(so, 0.0)
    mu = jnp.mean(so, axis=0, keepdims=True)
    var = jnp.mean((so - mu) ** 2, axis=0, keepdims=True)
    esT = (so - mu) / jnp.sqrt(var + 1e-5) * lgT[...] + lbT[...]
    vn2 = jnp.sqrt(vox * vox + voy * voy + voz * voz + 1e-8)
    e40 = jnp.concatenate([esT, vox / vn2, voy / vn2, voz / vn2,
                           jnp.zeros((5, EBLK), f32)], axis=0)
    out_ref[...] = e40.T


def _edge_call(geo, mu_col, fr_col, wts):
    in_specs = [
        pl.BlockSpec((1, 8, EBLK), lambda i: (i, 0, 0)),
        pl.BlockSpec((16, 1), lambda i: (0, 0)),
        pl.BlockSpec((8, 1), lambda i: (0, 0)),
    ] + [pl.BlockSpec(w.shape, lambda i, n=len(w.shape): (0,) * n) for w in wts]
    return pl.pallas_call(
        _edge_body,
        grid=(NEDGE // EBLK,),
        in_specs=in_specs,
        out_specs=pl.BlockSpec((EBLK, EW), lambda i: (i, 0)),
        out_shape=jax.ShapeDtypeStruct((NEDGE, EW), f32),
    )(geo, mu_col, fr_col, *wts)


def _node_body(sin_ref, vx_ref, vy_ref, vz_ref, Wh, WsS, WsV, bs, Wv, Wg, bg,
               lg, lb, out_ref):
    so, vox, voy, voz = _gvp_small(sin_ref[...], vx_ref[...], vy_ref[...],
                                   vz_ref[...], Wh[...], WsS[...], WsV[...],
                                   bs[...], Wv[...], Wg[...], bg[...],
                                   final=False)
    s, vx, vy, vz = _layernorm(so, vox, voy, voz, lg[...], lb[...])
    nb = s.shape[0]
    out_ref[:, 0:NSD] = s
    out_ref[:, 100:116] = vx
    out_ref[:, 116:132] = vy
    out_ref[:, 132:148] = vz
    out_ref[:, 148:TW] = jnp.zeros((nb, TW - 148), f32)


def _node_call(sin24, vfx, vfy, vfz, wts):
    blk = 512
    in_specs = [
        pl.BlockSpec((blk, 24), lambda i: (i, 0)),
        pl.BlockSpec((blk, 2), lambda i: (i, 0)),
        pl.BlockSpec((blk, 2), lambda i: (i, 0)),
        pl.BlockSpec((blk, 2), lambda i: (i, 0)),
    ] + [pl.BlockSpec(w.shape, lambda i, n=len(w.shape): (0,) * n) for w in wts]
    return pl.pallas_call(
        _node_body,
        grid=(NNODE // blk,),
        in_specs=in_specs,
        out_specs=pl.BlockSpec((blk, TW), lambda i: (i, 0)),
        out_shape=jax.ShapeDtypeStruct((NNODE, TW), f32),
    )(sin24, vfx, vfy, vfz, *wts)


def _final_body(o_ref, c_ref, s_ref, vr_ref):
    nodes = o_ref[...]
    s = nodes[:, 0:NSD]
    vx = nodes[:, 100:116]
    vy = nodes[:, 116:132]
    vz = nodes[:, 132:148]
    C = c_ref[...]
    v1x, v1y, v1z = C[:, 6:7] - C[:, 3:4], C[:, 7:8] - C[:, 4:5], C[:, 8:9] - C[:, 5:6]
    v2x, v2y, v2z = C[:, 0:1] - C[:, 3:4], C[:, 1:2] - C[:, 4:5], C[:, 2:3] - C[:, 5:6]
    n1 = jnp.sqrt(v1x * v1x + v1y * v1y + v1z * v1z + 1e-8)
    e1x, e1y, e1z = v1x / n1, v1y / n1, v1z / n1
    d12 = e1x * v2x + e1y * v2y + e1z * v2z
    u2x, u2y, u2z = v2x - e1x * d12, v2y - e1y * d12, v2z - e1z * d12
    n2 = jnp.sqrt(u2x * u2x + u2y * u2y + u2z * u2z + 1e-8)
    e2x, e2y, e2z = u2x / n2, u2y / n2, u2z / n2
    e3x = e1y * e2z - e1z * e2y
    e3y = e1z * e2x - e1x * e2z
    e3z = e1x * e2y - e1y * e2x
    s_ref[...] = s
    vr_ref[:, 0:16] = vx * e1x + vy * e1y + vz * e1z
    vr_ref[:, 16:32] = vx * e2x + vy * e2y + vz * e2z
    vr_ref[:, 32:48] = vx * e3x + vy * e3y + vz * e3z


def _final_call(o4, cflat):
    blk = 512
    return pl.pallas_call(
        _final_body,
        grid=(NNODE // blk,),
        in_specs=[
            pl.BlockSpec((blk, TW), lambda i: (i, 0)),
            pl.BlockSpec((blk, 16), lambda i: (i, 0)),
        ],
        out_specs=[
            pl.BlockSpec((blk, NSD), lambda i: (i, 0)),
            pl.BlockSpec((blk, 48), lambda i: (i, 0)),
        ],
        out_shape=[
            jax.ShapeDtypeStruct((NNODE, NSD), f32),
            jax.ShapeDtypeStruct((NNODE, 48), f32),
        ],
    )(o4, cflat)


def _norm_(x, axis=-1, keepdims=False):
    return jnp.sqrt(jnp.sum(x * x, axis=axis, keepdims=keepdims) + 1e-8)


def _normalize_(x, axis=-1):
    return x / _norm_(x, axis=axis, keepdims=True)


def _dih_features(coords):
    X = coords.reshape(coords.shape[0], -1, 3)
    dX = X[:, 1:] - X[:, :-1]
    U = _normalize_(dX)
    u2, u1, u0 = U[:, :-2], U[:, 1:-1], U[:, 2:]
    n2 = _normalize_(jnp.cross(u2, u1))
    n1 = _normalize_(jnp.cross(u1, u0))
    cosD = jnp.clip(jnp.sum(n2 * n1, axis=-1), -1 + 1e-7, 1 - 1e-7)
    D = jnp.sign(jnp.sum(u2 * n1, axis=-1)) * jnp.arccos(cosD)
    D = jnp.pad(D, ((0, 0), (1, 2)))
    D = D.reshape(D.shape[0], -1, 3)
    return jnp.concatenate([jnp.cos(D), jnp.sin(D)], axis=-1)


def _orient(ca):
    fwd = _normalize_(ca[:, 1:] - ca[:, :-1])
    bwd = _normalize_(ca[:, :-1] - ca[:, 1:])
    fwd = jnp.pad(fwd, ((0, 0), (0, 1), (0, 0)))
    bwd = jnp.pad(bwd, ((0, 0), (1, 0), (0, 0)))
    return jnp.stack([fwd, bwd], axis=-2)


def _zpad_rows(w, total, off):
    out = jnp.zeros((total, w.shape[1]), f32)
    return out.at[off:off + w.shape[0]].set(w)


def _conv_weights(p):
    m0, m1, m2, fp0, fp1 = p['m0'], p['m1'], p['m2'], p['f0'], p['f1']
    Wh0, Ws0 = m0['Wh'], m0['Ws']
    wts = [
        _zpad_rows(Ws0[0:100], TW, 0),
        _zpad_rows(Ws0[100:132], EW, 0),
        Ws0[132:232],
        Ws0[232:265],
        m0['bs'][None, :],
        _zpad_rows(Wh0[0:16], TW, 100),
        _zpad_rows(Wh0[0:16], TW, 116),
        _zpad_rows(Wh0[0:16], TW, 132),
        _zpad_rows(Wh0[16:17], EW, 32),
        _zpad_rows(Wh0[16:17], EW, 33),
        _zpad_rows(Wh0[16:17], EW, 34),
        Wh0[17:33],
        m0['Wv'], m0['Wg'], m0['bg'][None, :],
    ]
    for m in (m1, m2):
        wts += [m['Wh'], m['Ws'][0:100], m['Ws'][100:116], m['bs'][None, :],
                m['Wv'], m['Wg'], m['bg'][None, :]]
    wts += [p['ln1']['g'][None, :], p['ln1']['b'][None, :]]
    wts += [fp0['Wh'], fp0['Ws'][0:100], fp0['Ws'][100:132], fp0['bs'][None, :],
            fp0['Wv'], fp0['Wg'], fp0['bg'][None, :]]
    wts += [fp1['Wh'], fp1['Ws'][0:200], fp1['Ws'][200:232], fp1['bs'][None, :],
            fp1['Wv'], fp1['Wg'], fp1['bg'][None, :]]
    wts += [p['ln2']['g'][None, :], p['ln2']['b'][None, :]]
    return wts


def kernel(struc_seqs, coords, coord_mask, padding_mask, confidence, params):
    del struc_seqs, coord_mask, padding_mask
    coords = coords.astype(f32)
    ca = coords[:, :, 1, :]

    ca_rows = jnp.pad(ca, ((0, 0), (0, 0), (0, 5)))
    ca_cols = jnp.transpose(ca_rows, (0, 2, 1))
    ca_hi = ca_cols.astype(jnp.bfloat16).astype(f32)
    ca_mid = (ca_cols - ca_hi).astype(jnp.bfloat16).astype(f32)
    ca_lo = (ca_cols - ca_hi - ca_mid).astype(jnp.bfloat16).astype(f32)
    ca_splits = jnp.concatenate([ca_hi, ca_mid, ca_lo], axis=1)
    idxg, geo = _knn_call(ca_rows, ca_splits, ca_cols)
    src_idx = idxg.reshape(NEDGE)

    ep = params['embed_edge']
    mu_col = jnp.linspace(0.0, 20.0, 16, dtype=f32)[:, None]
    fr_col = jnp.exp(jnp.arange(0, 16, 2, dtype=f32) * (-np.log(10000.0) / 16))[:, None]
    e_wts = [ep['Ws'][0:16].T, ep['Ws'][16:24].T, ep['Ws'][24:32].T,
             ep['Ws'][32:33].T, ep['bs'][:, None], ep['Wh'], ep['Wv'],
             ep['Wg'].T, ep['bg'][None, :],
             params['ln_edge']['g'][:, None], params['ln_edge']['b'][:, None]]
    etab = _edge_call(geo, mu_col, fr_col, e_wts)

    dih = _dih_features(coords)
    mu_c = jnp.linspace(0.0, 1.0, 16, dtype=f32)
    conf = jnp.exp(-(((confidence[..., None] - mu_c) * 16.0) ** 2))
    sin = jnp.concatenate([dih, conf], axis=-1).reshape(NNODE, 22)
    sin24 = jnp.pad(sin, ((0, 0), (0, 2)))
    ori = _orient(ca).reshape(NNODE, 2, 3)
    vfx, vfy, vfz = ori[:, :, 0], ori[:, :, 1], ori[:, :, 2]
    npp = params['embed_node']
    n_wts = [npp['Wh'], _zpad_rows(npp['Ws'][0:22], 24, 0), npp['Ws'][22:38],
             npp['bs'][None, :], npp['Wv'], npp['Wg'], npp['bg'][None, :],
             params['ln_node']['g'][None, :], params['ln_node']['b'][None, :]]
    otab = _node_call(sin24, vfx, vfy, vfz, n_wts)

    for lp in params['layers']:
        gat = _sc_gather(otab, src_idx)
        otab = _conv_call(otab, gat, etab, _conv_weights(lp))

    cflat = jnp.pad(coords.reshape(NNODE, 9), ((0, 0), (0, 7)))
    s_out, vr = _final_call(otab, cflat)
    vrot = vr.reshape(NNODE, 3, NVD).transpose(0, 2, 1).reshape(NNODE, NVD * 3)
    return jnp.concatenate([s_out, vrot], axis=-1).reshape(BB, LL, NSD + NVD * 3)


# speedup vs baseline: 1.0112x; 1.0005x over previous
import functools

import jax
import jax.numpy as jnp
import numpy as np
from jax import lax
from jax.experimental import pallas as pl
from jax.experimental.pallas import tpu as pltpu
from jax.experimental.pallas import tpu_sc as plsc

BB, LL, KK = 4, 1024, 30
NSD, NVD = 100, 16
ESD, EVD = 32, 1
NLAY = 4
NNODE = BB * LL
NEDGE = NNODE * KK
TW = 256
EW = 40
NBLK = 128
EBLK = NBLK * KK
NGRID = NNODE // NBLK

f32 = jnp.float32


def _knn_body(cand_ref, qs_ref, cols_ref, idx_ref, geo_ref, ohsc, idsc):
    b = pl.program_id(0)
    j = pl.program_id(1)
    cand = cand_ref[0]
    q = cols_ref[0]
    cx, cy, cz = cand[:, 0:1], cand[:, 1:2], cand[:, 2:3]
    rx, ry, rz = q[0:1, :], q[1:2, :], q[2:3, :]
    x2c = cx * cx + cy * cy + cz * cz
    x2r = rx * rx + ry * ry + rz * rz
    dot = cx * rx + cy * ry + cz * rz
    d2 = (x2r + x2c) - 2.0 * dot
    d2 = jnp.maximum(d2, 0.0)
    cand_ids = jax.lax.broadcasted_iota(jnp.int32, (LL, NBLK), 0)
    row_ids = jax.lax.broadcasted_iota(jnp.int32, (LL, NBLK), 1) + j * NBLK
    d2 = jnp.where(cand_ids == row_ids, d2 + 1e12, d2)

    candf = jax.lax.broadcasted_iota(jnp.int32, (LL, 1), 0).astype(f32)
    rowf = (jax.lax.broadcasted_iota(jnp.int32, (1, NBLK), 1) + j * NBLK).astype(f32)
    val = d2
    for k in range(KK):
        m = jnp.min(val, axis=0, keepdims=True)
        eq = val == m
        idxf = jnp.min(jnp.where(eq, candf, 3e9), axis=0, keepdims=True)
        oh = candf == idxf
        ohsc[:, NBLK * k:NBLK * (k + 1)] = oh.astype(f32)
        idsc[0:1, NBLK * k:NBLK * (k + 1)] = idxf
        idx_ref[0, k:k + 1, :] = (idxf + jnp.float32(1024.0) * b.astype(f32)
                                  ).astype(jnp.int32)
        val = jnp.where(oh, 1e30, val)

    qs = qs_ref[0]
    caj24 = jnp.dot(qs, ohsc[...])
    cajT = caj24[0:8, :] + caj24[8:16, :] + caj24[16:24, :]
    geoT = cajT - jnp.tile(q, (1, KK))
    offs = idsc[...] - jnp.tile(rowf, (1, KK))
    sub8 = jax.lax.broadcasted_iota(jnp.int32, (8, EBLK), 0)
    geo_ref[0] = jnp.where(sub8 == 3, offs, geoT)


def _knn_call(ca_rows, ca_splits, ca_cols):
    return pl.pallas_call(
        _knn_body,
        grid=(BB, LL // NBLK),
        in_specs=[
            pl.BlockSpec((1, LL, 8), lambda b, j: (b, 0, 0)),
            pl.BlockSpec((1, 24, LL), lambda b, j: (b, 0, 0)),
            pl.BlockSpec((1, 8, NBLK), lambda b, j: (b, 0, j)),
        ],
        out_specs=[
            pl.BlockSpec((1, KK, NBLK), lambda b, j: (b * 8 + j, 0, 0)),
            pl.BlockSpec((1, 8, EBLK), lambda b, j: (b * 8 + j, 0, 0)),
        ],
        out_shape=[
            jax.ShapeDtypeStruct((NGRID, KK, NBLK), jnp.int32),
            jax.ShapeDtypeStruct((NGRID, 8, EBLK), f32),
        ],
        scratch_shapes=[
            pltpu.VMEM((LL, EBLK), f32),
            pltpu.VMEM((1, EBLK), f32),
        ],
    )(ca_rows, ca_splits, ca_cols)


_SC_NW = 32
_SC_BPW = NEDGE // _SC_NW
_SC_CH = 128
_SC_NCH = _SC_BPW // _SC_CH


def _sc_gather(table, idx):
    mesh = plsc.VectorSubcoreMesh(core_axis_name="c", subcore_axis_name="s")

    @functools.partial(
        pl.kernel,
        mesh=mesh,
        out_type=jax.ShapeDtypeStruct((NEDGE, TW), f32),
        scratch_types=[
            pltpu.VMEM((_SC_CH,), jnp.int32),
            pltpu.VMEM((_SC_CH,), jnp.int32),
            pltpu.VMEM((_SC_CH, TW), f32),
            pltpu.VMEM((_SC_CH, TW), f32),
            pltpu.SemaphoreType.DMA,
            pltpu.SemaphoreType.DMA,
        ],
    )
    def k(table_hbm, idx_hbm, out_hbm, idx0, idx1, rows0, rows1, sem0, sem1):
        wid = lax.axis_index("s") * 2 + lax.axis_index("c")
        base = wid * _SC_BPW

        @pl.loop(0, _SC_NCH, step=2)
        def _(c):
            off0 = base + c * _SC_CH
            off1 = off0 + _SC_CH
            pltpu.sync_copy(idx_hbm.at[pl.ds(off0, _SC_CH)], idx0)
            cp0 = pltpu.async_copy(table_hbm.at[idx0], rows0, sem0)
            pltpu.sync_copy(idx_hbm.at[pl.ds(off1, _SC_CH)], idx1)
            cp1 = pltpu.async_copy(table_hbm.at[idx1], rows1, sem1)
            cp0.wait()
            pltpu.sync_copy(rows0, out_hbm.at[pl.ds(off0, _SC_CH)])
            cp1.wait()
            pltpu.sync_copy(rows1, out_hbm.at[pl.ds(off1, _SC_CH)])

    return k(table, idx)


def _gvp_small(s_in, vx, vy, vz, Wh, WsS, WsV, bs, Wv, Wg, bg, final):
    vhx, vhy, vhz = jnp.dot(vx, Wh), jnp.dot(vy, Wh), jnp.dot(vz, Wh)
    vn = jnp.sqrt(vhx * vhx + vhy * vhy + vhz * vhz + 1e-8)
    so = jnp.dot(s_in, WsS) + jnp.dot(vn, WsV) + bs
    gate = jax.nn.sigmoid(jnp.dot(so, Wg) + bg)
    vox = jnp.dot(vhx, Wv) * gate
    voy = jnp.dot(vhy, Wv) * gate
    voz = jnp.dot(vhz, Wv) * gate
    if not final:
        so = jnp.maximum(so, 0.0)
    return so, vox, voy, voz


def _layernorm(s, vx, vy, vz, g, b):
    mu = jnp.mean(s, axis=-1, keepdims=True)
    var = jnp.mean((s - mu) ** 2, axis=-1, keepdims=True)
    s = (s - mu) / jnp.sqrt(var + 1e-5) * g + b
    vn = jnp.sqrt(jnp.mean(vx * vx + vy * vy + vz * vz, axis=-1, keepdims=True) + 1e-8)
    return s, vx / vn, vy / vn, vz / vn


def _conv_body(*refs):
    (o_ref, g_ref, e_ref,
     WsG, WsE, WsD, WsVn, bs0, WhGx, WhGy, WhGz, WhEx, WhEy, WhEz, WhD,
     Wv0, Wg0, bg0,
     Wh1, Ws1S, Ws1V, bs1, Wv1, Wg1, bg1,
     Wh2, Ws2S, Ws2V, bs2, Wv2, Wg2, bg2,
     g1, b1,
     Fh0, F0S, F0V, fb0, Fv0, Fg0, fg0,
     Fh1, F1S, F1V, fb1, Fv1, Fg1, fg1,
     g2, b2,
     out_ref) = refs

    nodes = o_ref[...]
    s = nodes[:, 0:NSD]
    nvx = nodes[:, 100:116]
    nvy = nodes[:, 116:132]
    nvz = nodes[:, 132:148]
    g = g_ref[...]
    e = e_ref[...]

    def rep(t):
        return jnp.broadcast_to(t[None], (KK,) + t.shape).reshape(EBLK, t.shape[-1])

    def kmean(x):
        return x.reshape(KK, NBLK, x.shape[-1]).sum(axis=0) / jnp.float32(KK)

    vhx = jnp.dot(g, WhGx[...]) + jnp.dot(e, WhEx[...]) + rep(jnp.dot(nvx, WhD[...]))
    vhy = jnp.dot(g, WhGy[...]) + jnp.dot(e, WhEy[...]) + rep(jnp.dot(nvy, WhD[...]))
    vhz = jnp.dot(g, WhGz[...]) + jnp.dot(e, WhEz[...]) + rep(jnp.dot(nvz, WhD[...]))
    vn = jnp.sqrt(vhx * vhx + vhy * vhy + vhz * vhz + 1e-8)
    so = (jnp.dot(g, WsG[...]) + jnp.dot(e, WsE[...])
          + rep(jnp.dot(s, WsD[...])) + jnp.dot(vn, WsVn[...]) + bs0[...])
    gate = jax.nn.sigmoid(jnp.dot(so, Wg0[...]) + bg0[...])
    mvx = jnp.dot(vhx, Wv0[...]) * gate
    mvy = jnp.dot(vhy, Wv0[...]) * gate
    mvz = jnp.dot(vhz, Wv0[...]) * gate
    ms = jnp.maximum(so, 0.0)

    ms, mvx, mvy, mvz = _gvp_small(ms, mvx, mvy, mvz, Wh1[...], Ws1S[...],
                                   Ws1V[...], bs1[...], Wv1[...], Wg1[...],
                                   bg1[...], final=False)
    ms, mvx, mvy, mvz = _gvp_small(ms, mvx, mvy, mvz, Wh2[...], Ws2S[...],
                                   Ws2V[...], bs2[...], Wv2[...], Wg2[...],
                                   bg2[...], final=True)

    ags = kmean(ms)
    agvx = kmean(mvx)
    agvy = kmean(mvy)
    agvz = kmean(mvz)

    s1, vx1, vy1, vz1 = _layernorm(s + ags, nvx + agvx, nvy + agvy, nvz + agvz,
                                   g1[...], b1[...])

    fs, fvx, fvy, fvz = _gvp_small(s1, vx1, vy1, vz1, Fh0[...], F0S[...],
                                   F0V[...], fb0[...], Fv0[...], Fg0[...],
                                   fg0[...], final=False)
    fs, fvx, fvy, fvz = _gvp_small(fs, fvx, fvy, fvz, Fh1[...], F1S[...],
                                   F1V[...], fb1[...], Fv1[...], Fg1[...],
                                   fg1[...], final=True)

    s2, vx2, vy2, vz2 = _layernorm(s1 + fs, vx1 + fvx, vy1 + fvy, vz1 + fvz,
                                   g2[...], b2[...])

    out_ref[:, 0:NSD] = s2
    out_ref[:, 100:116] = vx2
    out_ref[:, 116:132] = vy2
    out_ref[:, 132:148] = vz2
    out_ref[:, 148:TW] = jnp.zeros((NBLK, TW - 148), f32)


def _conv_call(o_prev, gat, etab, wts):
    in_specs = [
        pl.BlockSpec((NBLK, TW), lambda i: (i, 0)),
        pl.BlockSpec((EBLK, TW), lambda i: (i, 0)),
        pl.BlockSpec((EBLK, EW), lambda i: (i, 0)),
    ] + [pl.BlockSpec(w.shape, lambda i, n=len(w.shape): (0,) * n) for w in wts]
    return pl.pallas_call(
        _conv_body,
        grid=(NGRID,),
        in_specs=in_specs,
        out_specs=pl.BlockSpec((NBLK, TW), lambda i: (i, 0)),
        out_shape=jax.ShapeDtypeStruct((NNODE, TW), f32),
    )(o_prev, gat, etab, *wts)


def _edge_body(geo_ref, mu_ref, fr_ref, WsRT, WsCT, WsST, WsVrT, bsT, Whs,
               Wvs, WgT, bg, lgT, lbT, out_ref):
    g8 = geo_ref[0]
    dx, dy, dz, off = g8[0:1, :], g8[1:2, :], g8[2:3, :], g8[3:4, :]
    d2 = dx * dx + dy * dy + dz * dz
    dist = jnp.sqrt(d2 + 1e-8)
    rbfT = jnp.exp(-(((dist - mu_ref[...]) / 1.25) ** 2))
    angT = off * fr_ref[...]
    caT, saT = jnp.cos(angT), jnp.sin(angT)
    ex, ey, ez = dx / dist, dy / dist, dz / dist
    wh = Whs[...]
    vhx, vhy, vhz = ex * wh, ey * wh, ez * wh
    vn = jnp.sqrt(vhx * vhx + vhy * vhy + vhz * vhz + 1e-8)
    so = (jnp.dot(WsRT[...], rbfT) + jnp.dot(WsCT[...], caT)
          + jnp.dot(WsST[...], saT) + WsVrT[...] * vn + bsT[...])
    gate = jax.nn.sigmoid(jnp.dot(WgT[...], so) + bg[...])
    wv = Wvs[...]
    vox, voy, voz = vhx * wv * gate, vhy * wv * gate, vhz * wv * gate
    so = jnp.maximum(so, 0.0)
    mu = jnp.mean(so, axis=0, keepdims=True)
    var = jnp.mean((so - mu) ** 2, axis=0, keepdims=True)
    esT = (so - mu) / jnp.sqrt(var + 1e-5) * lgT[...] + lbT[...]
    vn2 = jnp.sqrt(vox * vox + voy * voy + voz * voz + 1e-8)
    e40 = jnp.concatenate([esT, vox / vn2, voy / vn2, voz / vn2,
                           jnp.zeros((5, EBLK), f32)], axis=0)
    out_ref[...] = e40.T


def _edge_call(geo, mu_col, fr_col, wts):
    in_specs = [
        pl.BlockSpec((1, 8, EBLK), lambda i: (i, 0, 0)),
        pl.BlockSpec((16, 1), lambda i: (0, 0)),
        pl.BlockSpec((8, 1), lambda i: (0, 0)),
    ] + [pl.BlockSpec(w.shape, lambda i, n=len(w.shape): (0,) * n) for w in wts]
    return pl.pallas_call(
        _edge_body,
        grid=(NEDGE // EBLK,),
        in_specs=in_specs,
        out_specs=pl.BlockSpec((EBLK, EW), lambda i: (i, 0)),
        out_shape=jax.ShapeDtypeStruct((NEDGE, EW), f32),
    )(geo, mu_col, fr_col, *wts)


def _node_body(sin_ref, vx_ref, vy_ref, vz_ref, Wh, WsS, WsV, bs, Wv, Wg, bg,
               lg, lb, out_ref):
    so, vox, voy, voz = _gvp_small(sin_ref[...], vx_ref[...], vy_ref[...],
                                   vz_ref[...], Wh[...], WsS[...], WsV[...],
                                   bs[...], Wv[...], Wg[...], bg[...],
                                   final=False)
    s, vx, vy, vz = _layernorm(so, vox, voy, voz, lg[...], lb[...])
    nb = s.shape[0]
    out_ref[:, 0:NSD] = s
    out_ref[:, 100:116] = vx
    out_ref[:, 116:132] = vy
    out_ref[:, 132:148] = vz
    out_ref[:, 148:TW] = jnp.zeros((nb, TW - 148), f32)


def _node_call(sin24, vfx, vfy, vfz, wts):
    blk = 512
    in_specs = [
        pl.BlockSpec((blk, 24), lambda i: (i, 0)),
        pl.BlockSpec((blk, 2), lambda i: (i, 0)),
        pl.BlockSpec((blk, 2), lambda i: (i, 0)),
        pl.BlockSpec((blk, 2), lambda i: (i, 0)),
    ] + [pl.BlockSpec(w.shape, lambda i, n=len(w.shape): (0,) * n) for w in wts]
    return pl.pallas_call(
        _node_body,
        grid=(NNODE // blk,),
        in_specs=in_specs,
        out_specs=pl.BlockSpec((blk, TW), lambda i: (i, 0)),
        out_shape=jax.ShapeDtypeStruct((NNODE, TW), f32),
    )(sin24, vfx, vfy, vfz, *wts)


def _final_body(o_ref, c_ref, s_ref, vr_ref):
    nodes = o_ref[...]
    s = nodes[:, 0:NSD]
    vx = nodes[:, 100:116]
    vy = nodes[:, 116:132]
    vz = nodes[:, 132:148]
    C = c_ref[...]
    v1x, v1y, v1z = C[:, 6:7] - C[:, 3:4], C[:, 7:8] - C[:, 4:5], C[:, 8:9] - C[:, 5:6]
    v2x, v2y, v2z = C[:, 0:1] - C[:, 3:4], C[:, 1:2] - C[:, 4:5], C[:, 2:3] - C[:, 5:6]
    n1 = jnp.sqrt(v1x * v1x + v1y * v1y + v1z * v1z + 1e-8)
    e1x, e1y, e1z = v1x / n1, v1y / n1, v1z / n1
    d12 = e1x * v2x + e1y * v2y + e1z * v2z
    u2x, u2y, u2z = v2x - e1x * d12, v2y - e1y * d12, v2z - e1z * d12
    n2 = jnp.sqrt(u2x * u2x + u2y * u2y + u2z * u2z + 1e-8)
    e2x, e2y, e2z = u2x / n2, u2y / n2, u2z / n2
    e3x = e1y * e2z - e1z * e2y
    e3y = e1z * e2x - e1x * e2z
    e3z = e1x * e2y - e1y * e2x
    s_ref[...] = s
    vr_ref[:, 0:16] = vx * e1x + vy * e1y + vz * e1z
    vr_ref[:, 16:32] = vx * e2x + vy * e2y + vz * e2z
    vr_ref[:, 32:48] = vx * e3x + vy * e3y + vz * e3z


def _final_call(o4, cflat):
    blk = 512
    return pl.pallas_call(
        _final_body,
        grid=(NNODE // blk,),
        in_specs=[
            pl.BlockSpec((blk, TW), lambda i: (i, 0)),
            pl.BlockSpec((blk, 16), lambda i: (i, 0)),
        ],
        out_specs=[
            pl.BlockSpec((blk, NSD), lambda i: (i, 0)),
            pl.BlockSpec((blk, 48), lambda i: (i, 0)),
        ],
        out_shape=[
            jax.ShapeDtypeStruct((NNODE, NSD), f32),
            jax.ShapeDtypeStruct((NNODE, 48), f32),
        ],
    )(o4, cflat)


def _norm_(x, axis=-1, keepdims=False):
    return jnp.sqrt(jnp.sum(x * x, axis=axis, keepdims=keepdims) + 1e-8)


def _normalize_(x, axis=-1):
    return x / _norm_(x, axis=axis, keepdims=True)


def _dih_features(coords):
    X = coords.reshape(coords.shape[0], -1, 3)
    dX = X[:, 1:] - X[:, :-1]
    U = _normalize_(dX)
    u2, u1, u0 = U[:, :-2], U[:, 1:-1], U[:, 2:]
    n2 = _normalize_(jnp.cross(u2, u1))
    n1 = _normalize_(jnp.cross(u1, u0))
    cosD = jnp.clip(jnp.sum(n2 * n1, axis=-1), -1 + 1e-7, 1 - 1e-7)
    D = jnp.sign(jnp.sum(u2 * n1, axis=-1)) * jnp.arccos(cosD)
    D = jnp.pad(D, ((0, 0), (1, 2)))
    D = D.reshape(D.shape[0], -1, 3)
    return jnp.concatenate([jnp.cos(D), jnp.sin(D)], axis=-1)


def _orient(ca):
    fwd = _normalize_(ca[:, 1:] - ca[:, :-1])
    bwd = _normalize_(ca[:, :-1] - ca[:, 1:])
    fwd = jnp.pad(fwd, ((0, 0), (0, 1), (0, 0)))
    bwd = jnp.pad(bwd, ((0, 0), (1, 0), (0, 0)))
    return jnp.stack([fwd, bwd], axis=-2)


def _zpad_rows(w, total, off):
    lo = jnp.zeros((off, w.shape[1]), f32)
    hi = jnp.zeros((total - off - w.shape[0], w.shape[1]), f32)
    return jnp.concatenate([lo, w, hi], axis=0)


def _conv_weights(p):
    m0, m1, m2, fp0, fp1 = p['m0'], p['m1'], p['m2'], p['f0'], p['f1']
    Wh0, Ws0 = m0['Wh'], m0['Ws']
    wts = [
        _zpad_rows(Ws0[0:100], TW, 0),
        _zpad_rows(Ws0[100:132], EW, 0),
        Ws0[132:232],
        Ws0[232:265],
        m0['bs'][None, :],
        _zpad_rows(Wh0[0:16], TW, 100),
        _zpad_rows(Wh0[0:16], TW, 116),
        _zpad_rows(Wh0[0:16], TW, 132),
        _zpad_rows(Wh0[16:17], EW, 32),
        _zpad_rows(Wh0[16:17], EW, 33),
        _zpad_rows(Wh0[16:17], EW, 34),
        Wh0[17:33],
        m0['Wv'], m0['Wg'], m0['bg'][None, :],
    ]
    for m in (m1, m2):
        wts += [m['Wh'], m['Ws'][0:100], m['Ws'][100:116], m['bs'][None, :],
                m['Wv'], m['Wg'], m['bg'][None, :]]
    wts += [p['ln1']['g'][None, :], p['ln1']['b'][None, :]]
    wts += [fp0['Wh'], fp0['Ws'][0:100], fp0['Ws'][100:132], fp0['bs'][None, :],
            fp0['Wv'], fp0['Wg'], fp0['bg'][None, :]]
    wts += [fp1['Wh'], fp1['Ws'][0:200], fp1['Ws'][200:232], fp1['bs'][None, :],
            fp1['Wv'], fp1['Wg'], fp1['bg'][None, :]]
    wts += [p['ln2']['g'][None, :], p['ln2']['b'][None, :]]
    return wts


def kernel(struc_seqs, coords, coord_mask, padding_mask, confidence, params):
    del struc_seqs, coord_mask, padding_mask
    coords = coords.astype(f32)
    ca = coords[:, :, 1, :]

    ca_rows = jnp.pad(ca, ((0, 0), (0, 0), (0, 5)))
    ca_cols = jnp.transpose(ca_rows, (0, 2, 1))
    ca_hi = ca_cols.astype(jnp.bfloat16).astype(f32)
    ca_mid = (ca_cols - ca_hi).astype(jnp.bfloat16).astype(f32)
    ca_lo = (ca_cols - ca_hi - ca_mid).astype(jnp.bfloat16).astype(f32)
    ca_splits = jnp.concatenate([ca_hi, ca_mid, ca_lo], axis=1)
    idxg, geo = _knn_call(ca_rows, ca_splits, ca_cols)
    src_idx = idxg.reshape(NEDGE)

    ep = params['embed_edge']
    mu_col = jnp.linspace(0.0, 20.0, 16, dtype=f32)[:, None]
    fr_col = jnp.exp(jnp.arange(0, 16, 2, dtype=f32) * (-np.log(10000.0) / 16))[:, None]
    e_wts = [ep['Ws'][0:16].T, ep['Ws'][16:24].T, ep['Ws'][24:32].T,
             ep['Ws'][32:33].T, ep['bs'][:, None], ep['Wh'], ep['Wv'],
             ep['Wg'].T, ep['bg'][None, :],
             params['ln_edge']['g'][:, None], params['ln_edge']['b'][:, None]]
    etab = _edge_call(geo, mu_col, fr_col, e_wts)

    dih = _dih_features(coords)
    mu_c = jnp.linspace(0.0, 1.0, 16, dtype=f32)
    conf = jnp.exp(-(((confidence[..., None] - mu_c) * 16.0) ** 2))
    sin = jnp.concatenate([dih, conf], axis=-1).reshape(NNODE, 22)
    sin24 = jnp.pad(sin, ((0, 0), (0, 2)))
    ori = _orient(ca).reshape(NNODE, 2, 3)
    vfx, vfy, vfz = ori[:, :, 0], ori[:, :, 1], ori[:, :, 2]
    npp = params['embed_node']
    n_wts = [npp['Wh'], _zpad_rows(npp['Ws'][0:22], 24, 0), npp['Ws'][22:38],
             npp['bs'][None, :], npp['Wv'], npp['Wg'], npp['bg'][None, :],
             params['ln_node']['g'][None, :], params['ln_node']['b'][None, :]]
    otab = _node_call(sin24, vfx, vfy, vfz, n_wts)

    for lp in params['layers']:
        gat = _sc_gather(otab, src_idx)
        otab = _conv_call(otab, gat, etab, _conv_weights(lp))

    cflat = jnp.pad(coords.reshape(NNODE, 9), ((0, 0), (0, 7)))
    s_out, vr = _final_call(otab, cflat)
    vrot = vr.reshape(NNODE, 3, NVD).transpose(0, 2, 1).reshape(NNODE, NVD * 3)
    return jnp.concatenate([s_out, vrot], axis=-1).reshape(BB, LL, NSD + NVD * 3)


# speedup vs baseline: 1.1036x; 1.0914x over previous
import functools

import jax
import jax.numpy as jnp
import numpy as np
from jax import lax
from jax.experimental import pallas as pl
from jax.experimental.pallas import tpu as pltpu
from jax.experimental.pallas import tpu_sc as plsc

BB, LL, KK = 4, 1024, 30
NSD, NVD = 100, 16
ESD, EVD = 32, 1
NLAY = 4
NNODE = BB * LL
NEDGE = NNODE * KK
TW = 256
EW = 40
NBLK = 256
EBLK = NBLK * KK
NGRID = NNODE // NBLK

f32 = jnp.float32


def _knn_body(cand_ref, qs_ref, cols_ref, idx_ref, geo_ref, ohsc, idsc):
    b = pl.program_id(0)
    j = pl.program_id(1)
    cand = cand_ref[0]
    q = cols_ref[0]
    cx, cy, cz = cand[:, 0:1], cand[:, 1:2], cand[:, 2:3]
    rx, ry, rz = q[0:1, :], q[1:2, :], q[2:3, :]
    x2c = cx * cx + cy * cy + cz * cz
    x2r = rx * rx + ry * ry + rz * rz
    dot = cx * rx + cy * ry + cz * rz
    d2 = (x2r + x2c) - 2.0 * dot
    d2 = jnp.maximum(d2, 0.0)
    cand_ids = jax.lax.broadcasted_iota(jnp.int32, (LL, NBLK), 0)
    row_ids = jax.lax.broadcasted_iota(jnp.int32, (LL, NBLK), 1) + j * NBLK
    d2 = jnp.where(cand_ids == row_ids, d2 + 1e12, d2)

    candf = jax.lax.broadcasted_iota(jnp.int32, (LL, 1), 0).astype(f32)
    rowf = (jax.lax.broadcasted_iota(jnp.int32, (1, NBLK), 1) + j * NBLK).astype(f32)
    val = d2
    for k in range(KK):
        m = jnp.min(val, axis=0, keepdims=True)
        eq = val == m
        idxf = jnp.min(jnp.where(eq, candf, 3e9), axis=0, keepdims=True)
        oh = candf == idxf
        ohsc[:, NBLK * k:NBLK * (k + 1)] = oh.astype(jnp.bfloat16)
        idsc[0:1, NBLK * k:NBLK * (k + 1)] = idxf
        idx_ref[0, k:k + 1, :] = (idxf + jnp.float32(1024.0) * b.astype(f32)
                                  ).astype(jnp.int32)
        val = jnp.where(oh, 1e30, val)

    qs = qs_ref[0]
    caj24 = jnp.dot(qs, ohsc[...], preferred_element_type=f32)
    cajT = caj24[0:8, :] + caj24[8:16, :] + caj24[16:24, :]
    geoT = cajT - jnp.tile(q, (1, KK))
    offs = idsc[...] - jnp.tile(rowf, (1, KK))
    sub8 = jax.lax.broadcasted_iota(jnp.int32, (8, EBLK), 0)
    geo_ref[0] = jnp.where(sub8 == 3, offs, geoT)


def _knn_call(ca_rows, ca_splits, ca_cols):
    return pl.pallas_call(
        _knn_body,
        grid=(BB, LL // NBLK),
        in_specs=[
            pl.BlockSpec((1, LL, 8), lambda b, j: (b, 0, 0)),
            pl.BlockSpec((1, 24, LL), lambda b, j: (b, 0, 0)),
            pl.BlockSpec((1, 8, NBLK), lambda b, j: (b, 0, j)),
        ],
        out_specs=[
            pl.BlockSpec((1, KK, NBLK),
                         lambda b, j: (b * (LL // NBLK) + j, 0, 0)),
            pl.BlockSpec((1, 8, EBLK),
                         lambda b, j: (b * (LL // NBLK) + j, 0, 0)),
        ],
        out_shape=[
            jax.ShapeDtypeStruct((NGRID, KK, NBLK), jnp.int32),
            jax.ShapeDtypeStruct((NGRID, 8, EBLK), f32),
        ],
        scratch_shapes=[
            pltpu.VMEM((LL, EBLK), jnp.bfloat16),
            pltpu.VMEM((1, EBLK), f32),
        ],
    )(ca_rows, ca_splits, ca_cols)


_SC_NW = 32
_SC_BPW = NEDGE // _SC_NW
_SC_CH = 128
_SC_NCH = _SC_BPW // _SC_CH


def _sc_gather(table, idx):
    mesh = plsc.VectorSubcoreMesh(core_axis_name="c", subcore_axis_name="s")

    @functools.partial(
        pl.kernel,
        mesh=mesh,
        out_type=jax.ShapeDtypeStruct((NEDGE, TW), f32),
        scratch_types=[
            pltpu.VMEM((_SC_CH,), jnp.int32),
            pltpu.VMEM((_SC_CH,), jnp.int32),
            pltpu.VMEM((_SC_CH, TW), f32),
            pltpu.VMEM((_SC_CH, TW), f32),
            pltpu.SemaphoreType.DMA,
            pltpu.SemaphoreType.DMA,
        ],
    )
    def k(table_hbm, idx_hbm, out_hbm, idx0, idx1, rows0, rows1, sem0, sem1):
        wid = lax.axis_index("s") * 2 + lax.axis_index("c")
        base = wid * _SC_BPW

        @pl.loop(0, _SC_NCH, step=2)
        def _(c):
            off0 = base + c * _SC_CH
            off1 = off0 + _SC_CH
            pltpu.sync_copy(idx_hbm.at[pl.ds(off0, _SC_CH)], idx0)
            cp0 = pltpu.async_copy(table_hbm.at[idx0], rows0, sem0)
            pltpu.sync_copy(idx_hbm.at[pl.ds(off1, _SC_CH)], idx1)
            cp1 = pltpu.async_copy(table_hbm.at[idx1], rows1, sem1)
            cp0.wait()
            pltpu.sync_copy(rows0, out_hbm.at[pl.ds(off0, _SC_CH)])
            cp1.wait()
            pltpu.sync_copy(rows1, out_hbm.at[pl.ds(off1, _SC_CH)])

    return k(table, idx)


def _gvp_small(s_in, vx, vy, vz, Wh, WsS, WsV, bs, Wv, Wg, bg, final):
    vhx, vhy, vhz = jnp.dot(vx, Wh), jnp.dot(vy, Wh), jnp.dot(vz, Wh)
    vn = jnp.sqrt(vhx * vhx + vhy * vhy + vhz * vhz + 1e-8)
    so = jnp.dot(s_in, WsS) + jnp.dot(vn, WsV) + bs
    gate = jax.nn.sigmoid(jnp.dot(so, Wg) + bg)
    vox = jnp.dot(vhx, Wv) * gate
    voy = jnp.dot(vhy, Wv) * gate
    voz = jnp.dot(vhz, Wv) * gate
    if not final:
        so = jnp.maximum(so, 0.0)
    return so, vox, voy, voz


def _layernorm(s, vx, vy, vz, g, b):
    mu = jnp.mean(s, axis=-1, keepdims=True)
    var = jnp.mean((s - mu) ** 2, axis=-1, keepdims=True)
    s = (s - mu) / jnp.sqrt(var + 1e-5) * g + b
    vn = jnp.sqrt(jnp.mean(vx * vx + vy * vy + vz * vz, axis=-1, keepdims=True) + 1e-8)
    return s, vx / vn, vy / vn, vz / vn


def _conv_body(*refs):
    (o_ref, g_ref, e_ref,
     WsG, WsE, WsD, WsVn, bs0, WhGx, WhGy, WhGz, WhEx, WhEy, WhEz, WhD,
     Wv0, Wg0, bg0,
     Wh1, Ws1S, Ws1V, bs1, Wv1, Wg1, bg1,
     Wh2, Ws2S, Ws2V, bs2, Wv2, Wg2, bg2,
     g1, b1,
     Fh0, F0S, F0V, fb0, Fv0, Fg0, fg0,
     Fh1, F1S, F1V, fb1, Fv1, Fg1, fg1,
     g2, b2,
     out_ref) = refs

    nodes = o_ref[...]
    s = nodes[:, 0:NSD]
    nvx = nodes[:, 100:116]
    nvy = nodes[:, 116:132]
    nvz = nodes[:, 132:148]
    g = g_ref[...]
    e = e_ref[...]

    def rep(t):
        return jnp.broadcast_to(t[None], (KK,) + t.shape).reshape(EBLK, t.shape[-1])

    def kmean(x):
        return x.reshape(KK, NBLK, x.shape[-1]).sum(axis=0) / jnp.float32(KK)

    vhx = jnp.dot(g, WhGx[...]) + jnp.dot(e, WhEx[...]) + rep(jnp.dot(nvx, WhD[...]))
    vhy = jnp.dot(g, WhGy[...]) + jnp.dot(e, WhEy[...]) + rep(jnp.dot(nvy, WhD[...]))
    vhz = jnp.dot(g, WhGz[...]) + jnp.dot(e, WhEz[...]) + rep(jnp.dot(nvz, WhD[...]))
    vn = jnp.sqrt(vhx * vhx + vhy * vhy + vhz * vhz + 1e-8)
    so = (jnp.dot(g, WsG[...]) + jnp.dot(e, WsE[...])
          + rep(jnp.dot(s, WsD[...])) + jnp.dot(vn, WsVn[...]) + bs0[...])
    gate = jax.nn.sigmoid(jnp.dot(so, Wg0[...]) + bg0[...])
    mvx = jnp.dot(vhx, Wv0[...]) * gate
    mvy = jnp.dot(vhy, Wv0[...]) * gate
    mvz = jnp.dot(vhz, Wv0[...]) * gate
    ms = jnp.maximum(so, 0.0)

    ms, mvx, mvy, mvz = _gvp_small(ms, mvx, mvy, mvz, Wh1[...], Ws1S[...],
                                   Ws1V[...], bs1[...], Wv1[...], Wg1[...],
                                   bg1[...], final=False)
    ms, mvx, mvy, mvz = _gvp_small(ms, mvx, mvy, mvz, Wh2[...], Ws2S[...],
                                   Ws2V[...], bs2[...], Wv2[...], Wg2[...],
                                   bg2[...], final=True)

    ags = kmean(ms)
    agvx = kmean(mvx)
    agvy = kmean(mvy)
    agvz = kmean(mvz)

    s1, vx1, vy1, vz1 = _layernorm(s + ags, nvx + agvx, nvy + agvy, nvz + agvz,
                                   g1[...], b1[...])

    fs, fvx, fvy, fvz = _gvp_small(s1, vx1, vy1, vz1, Fh0[...], F0S[...],
                                   F0V[...], fb0[...], Fv0[...], Fg0[...],
                                   fg0[...], final=False)
    fs, fvx, fvy, fvz = _gvp_small(fs, fvx, fvy, fvz, Fh1[...], F1S[...],
                                   F1V[...], fb1[...], Fv1[...], Fg1[...],
                                   fg1[...], final=True)

    s2, vx2, vy2, vz2 = _layernorm(s1 + fs, vx1 + fvx, vy1 + fvy, vz1 + fvz,
                                   g2[...], b2[...])

    out_ref[:, 0:NSD] = s2
    out_ref[:, 100:116] = vx2
    out_ref[:, 116:132] = vy2
    out_ref[:, 132:148] = vz2
    out_ref[:, 148:TW] = jnp.zeros((NBLK, TW - 148), f32)


def _conv_call(o_prev, gat, etab, wts):
    in_specs = [
        pl.BlockSpec((NBLK, TW), lambda i: (i, 0)),
        pl.BlockSpec((EBLK, TW), lambda i: (i, 0)),
        pl.BlockSpec((EBLK, EW), lambda i: (i, 0)),
    ] + [pl.BlockSpec(w.shape, lambda i, n=len(w.shape): (0,) * n) for w in wts]
    return pl.pallas_call(
        _conv_body,
        grid=(NGRID,),
        in_specs=in_specs,
        out_specs=pl.BlockSpec((NBLK, TW), lambda i: (i, 0)),
        out_shape=jax.ShapeDtypeStruct((NNODE, TW), f32),
    )(o_prev, gat, etab, *wts)


def _edge_body(geo_ref, mu_ref, fr_ref, WsRT, WsCT, WsST, WsVrT, bsT, Whs,
               Wvs, WgT, bg, lgT, lbT, out_ref):
    g8 = geo_ref[0]
    dx, dy, dz, off = g8[0:1, :], g8[1:2, :], g8[2:3, :], g8[3:4, :]
    d2 = dx * dx + dy * dy + dz * dz
    dist = jnp.sqrt(d2 + 1e-8)
    rbfT = jnp.exp(-(((dist - mu_ref[...]) / 1.25) ** 2))
    angT = off * fr_ref[...]
    caT, saT = jnp.cos(angT), jnp.sin(angT)
    ex, ey, ez = dx / dist, dy / dist, dz / dist
    wh = Whs[...]
    vhx, vhy, vhz = ex * wh, ey * wh, ez * wh
    vn = jnp.sqrt(vhx * vhx + vhy * vhy + vhz * vhz + 1e-8)
    so = (jnp.dot(WsRT[...], rbfT) + jnp.dot(WsCT[...], caT)
          + jnp.dot(WsST[...], saT) + WsVrT[...] * vn + bsT[...])
    gate = jax.nn.sigmoid(jnp.dot(WgT[...], so) + bg[...])
    wv = Wvs[...]
    vox, voy, voz = vhx * wv * gate, vhy * wv * gate, vhz * wv * gate
    so = jnp.maximum(so, 0.0)
    mu = jnp.mean(so, axis=0, keepdims=True)
    var = jnp.mean((so - mu) ** 2, axis=0, keepdims=True)
    esT = (so - mu) / jnp.sqrt(var + 1e-5) * lgT[...] + lbT[...]
    vn2 = jnp.sqrt(vox * vox + voy * voy + voz * voz + 1e-8)
    e40 = jnp.concatenate([esT, vox / vn2, voy / vn2, voz / vn2,
                           jnp.zeros((5, EBLK), f32)], axis=0)
    out_ref[...] = e40.T


def _edge_call(geo, mu_col, fr_col, wts):
    in_specs = [
        pl.BlockSpec((1, 8, EBLK), lambda i: (i, 0, 0)),
        pl.BlockSpec((16, 1), lambda i: (0, 0)),
        pl.BlockSpec((8, 1), lambda i: (0, 0)),
    ] + [pl.BlockSpec(w.shape, lambda i, n=len(w.shape): (0,) * n) for w in wts]
    return pl.pallas_call(
        _edge_body,
        grid=(NEDGE // EBLK,),
        in_specs=in_specs,
        out_specs=pl.BlockSpec((EBLK, EW), lambda i: (i, 0)),
        out_shape=jax.ShapeDtypeStruct((NEDGE, EW), f32),
    )(geo, mu_col, fr_col, *wts)


def _node_body(sin_ref, vx_ref, vy_ref, vz_ref, Wh, WsS, WsV, bs, Wv, Wg, bg,
               lg, lb, out_ref):
    so, vox, voy, voz = _gvp_small(sin_ref[...], vx_ref[...], vy_ref[...],
                                   vz_ref[...], Wh[...], WsS[...], WsV[...],
                                   bs[...], Wv[...], Wg[...], bg[...],
                                   final=False)
    s, vx, vy, vz = _layernorm(so, vox, voy, voz, lg[...], lb[...])
    nb = s.shape[0]
    out_ref[:, 0:NSD] = s
    out_ref[:, 100:116] = vx
    out_ref[:, 116:132] = vy
    out_ref[:, 132:148] = vz
    out_ref[:, 148:TW] = jnp.zeros((nb, TW - 148), f32)


def _node_call(sin24, vfx, vfy, vfz, wts):
    blk = 512
    in_specs = [
        pl.BlockSpec((blk, 24), lambda i: (i, 0)),
        pl.BlockSpec((blk, 2), lambda i: (i, 0)),
        pl.BlockSpec((blk, 2), lambda i: (i, 0)),
        pl.BlockSpec((blk, 2), lambda i: (i, 0)),
    ] + [pl.BlockSpec(w.shape, lambda i, n=len(w.shape): (0,) * n) for w in wts]
    return pl.pallas_call(
        _node_body,
        grid=(NNODE // blk,),
        in_specs=in_specs,
        out_specs=pl.BlockSpec((blk, TW), lambda i: (i, 0)),
        out_shape=jax.ShapeDtypeStruct((NNODE, TW), f32),
    )(sin24, vfx, vfy, vfz, *wts)


def _final_body(o_ref, c_ref, s_ref, vr_ref):
    nodes = o_ref[...]
    s = nodes[:, 0:NSD]
    vx = nodes[:, 100:116]
    vy = nodes[:, 116:132]
    vz = nodes[:, 132:148]
    C = c_ref[...]
    v1x, v1y, v1z = C[:, 6:7] - C[:, 3:4], C[:, 7:8] - C[:, 4:5], C[:, 8:9] - C[:, 5:6]
    v2x, v2y, v2z = C[:, 0:1] - C[:, 3:4], C[:, 1:2] - C[:, 4:5], C[:, 2:3] - C[:, 5:6]
    n1 = jnp.sqrt(v1x * v1x + v1y * v1y + v1z * v1z + 1e-8)
    e1x, e1y, e1z = v1x / n1, v1y / n1, v1z / n1
    d12 = e1x * v2x + e1y * v2y + e1z * v2z
    u2x, u2y, u2z = v2x - e1x * d12, v2y - e1y * d12, v2z - e1z * d12
    n2 = jnp.sqrt(u2x * u2x + u2y * u2y + u2z * u2z + 1e-8)
    e2x, e2y, e2z = u2x / n2, u2y / n2, u2z / n2
    e3x = e1y * e2z - e1z * e2y
    e3y = e1z * e2x - e1x * e2z
    e3z = e1x * e2y - e1y * e2x
    s_ref[...] = s
    vr_ref[:, 0:16] = vx * e1x + vy * e1y + vz * e1z
    vr_ref[:, 16:32] = vx * e2x + vy * e2y + vz * e2z
    vr_ref[:, 32:48] = vx * e3x + vy * e3y + vz * e3z


def _final_call(o4, cflat):
    blk = 512
    return pl.pallas_call(
        _final_body,
        grid=(NNODE // blk,),
        in_specs=[
            pl.BlockSpec((blk, TW), lambda i: (i, 0)),
            pl.BlockSpec((blk, 16), lambda i: (i, 0)),
        ],
        out_specs=[
            pl.BlockSpec((blk, NSD), lambda i: (i, 0)),
            pl.BlockSpec((blk, 48), lambda i: (i, 0)),
        ],
        out_shape=[
            jax.ShapeDtypeStruct((NNODE, NSD), f32),
            jax.ShapeDtypeStruct((NNODE, 48), f32),
        ],
    )(o4, cflat)


def _norm_(x, axis=-1, keepdims=False):
    return jnp.sqrt(jnp.sum(x * x, axis=axis, keepdims=keepdims) + 1e-8)


def _normalize_(x, axis=-1):
    return x / _norm_(x, axis=axis, keepdims=True)


def _dih_features(coords):
    X = coords.reshape(coords.shape[0], -1, 3)
    dX = X[:, 1:] - X[:, :-1]
    U = _normalize_(dX)
    u2, u1, u0 = U[:, :-2], U[:, 1:-1], U[:, 2:]
    n2 = _normalize_(jnp.cross(u2, u1))
    n1 = _normalize_(jnp.cross(u1, u0))
    cosD = jnp.clip(jnp.sum(n2 * n1, axis=-1), -1 + 1e-7, 1 - 1e-7)
    D = jnp.sign(jnp.sum(u2 * n1, axis=-1)) * jnp.arccos(cosD)
    D = jnp.pad(D, ((0, 0), (1, 2)))
    D = D.reshape(D.shape[0], -1, 3)
    return jnp.concatenate([jnp.cos(D), jnp.sin(D)], axis=-1)


def _orient(ca):
    fwd = _normalize_(ca[:, 1:] - ca[:, :-1])
    bwd = _normalize_(ca[:, :-1] - ca[:, 1:])
    fwd = jnp.pad(fwd, ((0, 0), (0, 1), (0, 0)))
    bwd = jnp.pad(bwd, ((0, 0), (1, 0), (0, 0)))
    return jnp.stack([fwd, bwd], axis=-2)


def _zpad_rows(w, total, off):
    lo = jnp.zeros((off, w.shape[1]), f32)
    hi = jnp.zeros((total - off - w.shape[0], w.shape[1]), f32)
    return jnp.concatenate([lo, w, hi], axis=0)


def _conv_weights(p):
    m0, m1, m2, fp0, fp1 = p['m0'], p['m1'], p['m2'], p['f0'], p['f1']
    Wh0, Ws0 = m0['Wh'], m0['Ws']
    wts = [
        _zpad_rows(Ws0[0:100], TW, 0),
        _zpad_rows(Ws0[100:132], EW, 0),
        Ws0[132:232],
        Ws0[232:265],
        m0['bs'][None, :],
        _zpad_rows(Wh0[0:16], TW, 100),
        _zpad_rows(Wh0[0:16], TW, 116),
        _zpad_rows(Wh0[0:16], TW, 132),
        _zpad_rows(Wh0[16:17], EW, 32),
        _zpad_rows(Wh0[16:17], EW, 33),
        _zpad_rows(Wh0[16:17], EW, 34),
        Wh0[17:33],
        m0['Wv'], m0['Wg'], m0['bg'][None, :],
    ]
    for m in (m1, m2):
        wts += [m['Wh'], m['Ws'][0:100], m['Ws'][100:116], m['bs'][None, :],
                m['Wv'], m['Wg'], m['bg'][None, :]]
    wts += [p['ln1']['g'][None, :], p['ln1']['b'][None, :]]
    wts += [fp0['Wh'], fp0['Ws'][0:100], fp0['Ws'][100:132], fp0['bs'][None, :],
            fp0['Wv'], fp0['Wg'], fp0['bg'][None, :]]
    wts += [fp1['Wh'], fp1['Ws'][0:200], fp1['Ws'][200:232], fp1['bs'][None, :],
            fp1['Wv'], fp1['Wg'], fp1['bg'][None, :]]
    wts += [p['ln2']['g'][None, :], p['ln2']['b'][None, :]]
    return wts


def kernel(struc_seqs, coords, coord_mask, padding_mask, confidence, params):
    del struc_seqs, coord_mask, padding_mask
    coords = coords.astype(f32)
    ca = coords[:, :, 1, :]

    ca_rows = jnp.pad(ca, ((0, 0), (0, 0), (0, 5)))
    ca_cols = jnp.transpose(ca_rows, (0, 2, 1))
    ca_hi = ca_cols.astype(jnp.bfloat16).astype(f32)
    ca_mid = (ca_cols - ca_hi).astype(jnp.bfloat16).astype(f32)
    ca_lo = (ca_cols - ca_hi - ca_mid).astype(jnp.bfloat16).astype(f32)
    ca_splits = jnp.concatenate([ca_hi, ca_mid, ca_lo],
                                axis=1).astype(jnp.bfloat16)
    idxg, geo = _knn_call(ca_rows, ca_splits, ca_cols)
    src_idx = idxg.reshape(NEDGE)

    ep = params['embed_edge']
    mu_col = jnp.linspace(0.0, 20.0, 16, dtype=f32)[:, None]
    fr_col = jnp.exp(jnp.arange(0, 16, 2, dtype=f32) * (-np.log(10000.0) / 16))[:, None]
    e_wts = [ep['Ws'][0:16].T, ep['Ws'][16:24].T, ep['Ws'][24:32].T,
             ep['Ws'][32:33].T, ep['bs'][:, None], ep['Wh'], ep['Wv'],
             ep['Wg'].T, ep['bg'][None, :],
             params['ln_edge']['g'][:, None], params['ln_edge']['b'][:, None]]
    etab = _edge_call(geo, mu_col, fr_col, e_wts)

    dih = _dih_features(coords)
    mu_c = jnp.linspace(0.0, 1.0, 16, dtype=f32)
    conf = jnp.exp(-(((confidence[..., None] - mu_c) * 16.0) ** 2))
    sin = jnp.concatenate([dih, conf], axis=-1).reshape(NNODE, 22)
    sin24 = jnp.pad(sin, ((0, 0), (0, 2)))
    ori = _orient(ca).reshape(NNODE, 2, 3)
    vfx, vfy, vfz = ori[:, :, 0], ori[:, :, 1], ori[:, :, 2]
    npp = params['embed_node']
    n_wts = [npp['Wh'], _zpad_rows(npp['Ws'][0:22], 24, 0), npp['Ws'][22:38],
             npp['bs'][None, :], npp['Wv'], npp['Wg'], npp['bg'][None, :],
             params['ln_node']['g'][None, :], params['ln_node']['b'][None, :]]
    otab = _node_call(sin24, vfx, vfy, vfz, n_wts)

    for lp in params['layers']:
        gat = _sc_gather(otab, src_idx)
        otab = _conv_call(otab, gat, etab, _conv_weights(lp))

    cflat = jnp.pad(coords.reshape(NNODE, 9), ((0, 0), (0, 7)))
    s_out, vr = _final_call(otab, cflat)
    vrot = vr.reshape(NNODE, 3, NVD).transpose(0, 2, 1).reshape(NNODE, NVD * 3)
    return jnp.concatenate([s_out, vrot], axis=-1).reshape(BB, LL, NSD + NVD * 3)
